# baseline (speedup 1.0000x reference)
import functools

import jax
import jax.numpy as jnp
from jax import lax
from jax.experimental import pallas as pl
from jax.experimental.pallas import tpu as pltpu
from jax.experimental.pallas import tpu_sc as plsc

N = 10000
E = 320000
H = 128
G = 64
HEADS = 4
DH = 32

NC = 2
NS = 16
NW = NC * NS
CH = 128
PTE = 10240
NCHUNK = PTE // CH
EP = NW * PTE
NP = 10112
STRIPE = NP // NS
KF_S, KS_S = 160, 0
KF_G, KS_G = 178, 142
NBLK = 10
BLK = N // NBLK

f32 = jnp.float32
i32 = jnp.int32

_mesh = plsc.VectorSubcoreMesh(core_axis_name="c", subcore_axis_name="s")
_untiled = pltpu.CompilerParams(use_tc_tiling_on_sc=False)


def _tile_ids():
    cid = lax.axis_index("c")
    sid = lax.axis_index("s")
    return cid, sid, cid * NS + sid


@functools.partial(
    pl.kernel,
    out_type=jax.ShapeDtypeStruct((NC, NP, 16), f32),
    mesh=_mesh,
    scratch_types=[
        pltpu.VMEM((CH,), i32),
        pltpu.VMEM((CH, 16), f32),
        pltpu.VMEM_SHARED((NP, 16), f32),
        pltpu.SemaphoreType.DMA,
    ],
)
def _sdeg(dp_hbm, ones_hbm, zeros_hbm, out_hbm, didx_v, ones_v, acc_sh, sem):
    cid, sid, wid = _tile_ids()
    pltpu.sync_copy(zeros_hbm.at[pl.ds(sid * STRIPE, STRIPE)],
                    acc_sh.at[pl.ds(sid * STRIPE, STRIPE)])
    pltpu.sync_copy(ones_hbm, ones_v)
    plsc.subcore_barrier()
    base = wid * PTE

    def step(j, c):
        pltpu.sync_copy(dp_hbm.at[pl.ds(base + j * CH, CH)], didx_v)
        pltpu.sync_copy(ones_v, acc_sh.at[didx_v], add=True)
        return c

    lax.fori_loop(0, NCHUNK, step, 0)
    plsc.subcore_barrier()
    pltpu.sync_copy(acc_sh.at[pl.ds(sid * STRIPE, STRIPE)],
                    out_hbm.at[cid, pl.ds(sid * STRIPE, STRIPE)])


@functools.partial(
    pl.kernel,
    out_type=jax.ShapeDtypeStruct((NC, NP, H), f32),
    mesh=_mesh,
    scratch_types=[
        pltpu.VMEM((CH,), i32),
        pltpu.VMEM((CH,), i32),
        pltpu.VMEM((CH,), i32),
        pltpu.VMEM((CH,), i32),
        pltpu.VMEM((CH, H), f32),
        pltpu.VMEM((CH, H), f32),
        pltpu.VMEM_SHARED((NP, H), f32),
        pltpu.SemaphoreType.DMA,
        pltpu.SemaphoreType.DMA,
    ],
)
def _sscat(q_hbm, sp_hbm, dp_hbm, zeros_hbm, out_hbm,
           sidx_a, sidx_b, didx_a, didx_b, rows_a, rows_b,
           acc_sh, sem_a, sem_b):
    cid, sid, wid = _tile_ids()
    pltpu.sync_copy(zeros_hbm.at[pl.ds(sid * STRIPE, STRIPE)],
                    acc_sh.at[pl.ds(sid * STRIPE, STRIPE)])
    plsc.subcore_barrier()
    base = jnp.where(cid == 0, sid * KF_S, NS * KF_S + sid * KS_S) * CH
    nch = jnp.where(cid == 0, KF_S, KS_S)

    @pl.when(nch > 0)
    def _():
        pltpu.sync_copy(sp_hbm.at[pl.ds(base, CH)], sidx_a)
        pltpu.sync_copy(dp_hbm.at[pl.ds(base, CH)], didx_a)
        pltpu.async_copy(q_hbm.at[sidx_a], rows_a, sem_a)

    def step(j2, c):
        o = base + 2 * j2 * CH
        pltpu.sync_copy(sp_hbm.at[pl.ds(o + CH, CH)], sidx_b)
        pltpu.sync_copy(dp_hbm.at[pl.ds(o + CH, CH)], didx_b)
        pltpu.make_async_copy(q_hbm.at[sidx_a], rows_a, sem_a).wait()
        pltpu.async_copy(q_hbm.at[sidx_b], rows_b, sem_b)
        pltpu.sync_copy(rows_a, acc_sh.at[didx_a], add=True)

        @pl.when(2 * j2 + 2 < nch)
        def _():
            pltpu.sync_copy(sp_hbm.at[pl.ds(o + 2 * CH, CH)], sidx_a)
            pltpu.sync_copy(dp_hbm.at[pl.ds(o + 2 * CH, CH)], didx_a)

        pltpu.make_async_copy(q_hbm.at[sidx_b], rows_b, sem_b).wait()

        @pl.when(2 * j2 + 2 < nch)
        def _():
            pltpu.async_copy(q_hbm.at[sidx_a], rows_a, sem_a)

        pltpu.sync_copy(rows_b, acc_sh.at[didx_b], add=True)
        return c

    lax.fori_loop(0, nch // 2, step, 0)
    plsc.subcore_barrier()
    pltpu.sync_copy(acc_sh.at[pl.ds(sid * STRIPE, STRIPE)],
                    out_hbm.at[cid, pl.ds(sid * STRIPE, STRIPE)])


@functools.partial(
    pl.kernel,
    out_type=(jax.ShapeDtypeStruct((EP, 16), f32),
              jax.ShapeDtypeStruct((NC, NP, 16), f32)),
    mesh=_mesh,
    compiler_params=_untiled,
    scratch_types=[
        pltpu.VMEM((CH,), i32),
        pltpu.VMEM((CH,), i32),
        pltpu.VMEM((CH,), i32),
        pltpu.VMEM((CH,), i32),
        pltpu.VMEM((CH, 16), f32),
        pltpu.VMEM((CH, 16), f32),
        pltpu.VMEM((CH, 16), f32),
        pltpu.VMEM((CH, 16), f32),
        pltpu.VMEM((CH, 16), f32),
        pltpu.VMEM((1, 16), f32),
        pltpu.VMEM_SHARED((NP, 16), f32),
        pltpu.SemaphoreType.DMA,
        pltpu.SemaphoreType.DMA,
    ],
)
def _satt(as_hbm, ad_hbm, m_hbm, sp_hbm, dp_hbm, zeros_hbm,
          ee_hbm, dens_hbm,
          sidx_a, sidx_b, didx_a, didx_b, as_a, as_b, ad_a, ad_b,
          ee_v, m_v, den_sh, sem_a, sem_b):
    cid, sid, wid = _tile_ids()
    pltpu.sync_copy(zeros_hbm.at[pl.ds(sid * STRIPE, STRIPE)],
                    den_sh.at[pl.ds(sid * STRIPE, STRIPE)])
    pltpu.sync_copy(m_hbm, m_v)
    plsc.subcore_barrier()
    base = wid * PTE
    pltpu.sync_copy(sp_hbm.at[pl.ds(base, CH)], sidx_a)
    pltpu.sync_copy(dp_hbm.at[pl.ds(base, CH)], didx_a)
    pltpu.async_copy(as_hbm.at[sidx_a], as_a, sem_a)
    pltpu.async_copy(ad_hbm.at[didx_a], ad_a, sem_a)

    def compute(as_v, ad_v):
        def inner(i, cc):
            m16 = m_v[0, :]
            a = as_v[i, :] + ad_v[i, :]
            ee_v[i, :] = jnp.exp(jnp.maximum(a, 0.2 * a) - m16)
            return cc

        lax.fori_loop(0, CH, inner, 0)

    def step(j2, c):
        off = base + 2 * j2 * CH
        pltpu.sync_copy(sp_hbm.at[pl.ds(off + CH, CH)], sidx_b)
        pltpu.sync_copy(dp_hbm.at[pl.ds(off + CH, CH)], didx_b)
        pltpu.make_async_copy(as_hbm.at[sidx_a], as_a, sem_a).wait()
        pltpu.make_async_copy(ad_hbm.at[didx_a], ad_a, sem_a).wait()
        pltpu.async_copy(as_hbm.at[sidx_b], as_b, sem_b)
        pltpu.async_copy(ad_hbm.at[didx_b], ad_b, sem_b)
        compute(as_a, ad_a)
        pltpu.sync_copy(ee_v, ee_hbm.at[pl.ds(off, CH)])
        pltpu.sync_copy(ee_v, den_sh.at[didx_a], add=True)

        @pl.when(2 * j2 + 2 < NCHUNK)
        def _():
            pltpu.sync_copy(sp_hbm.at[pl.ds(off + 2 * CH, CH)], sidx_a)
            pltpu.sync_copy(dp_hbm.at[pl.ds(off + 2 * CH, CH)], didx_a)

        pltpu.make_async_copy(as_hbm.at[sidx_b], as_b, sem_b).wait()
        pltpu.make_async_copy(ad_hbm.at[didx_b], ad_b, sem_b).wait()

        @pl.when(2 * j2 + 2 < NCHUNK)
        def _():
            pltpu.async_copy(as_hbm.at[sidx_a], as_a, sem_a)
            pltpu.async_copy(ad_hbm.at[didx_a], ad_a, sem_a)

        compute(as_b, ad_b)
        pltpu.sync_copy(ee_v, ee_hbm.at[pl.ds(off + CH, CH)])
        pltpu.sync_copy(ee_v, den_sh.at[didx_b], add=True)
        return c

    lax.fori_loop(0, NCHUNK // 2, step, 0)
    plsc.subcore_barrier()
    pltpu.sync_copy(den_sh.at[pl.ds(sid * STRIPE, STRIPE)],
                    dens_hbm.at[cid, pl.ds(sid * STRIPE, STRIPE)])


CHG = 64
NCHG = PTE // CHG


@functools.partial(
    pl.kernel,
    out_type=jax.ShapeDtypeStruct((NC, NP, H), f32),
    mesh=_mesh,
    scratch_types=[
        pltpu.VMEM((CHG,), i32),
        pltpu.VMEM((CHG,), i32),
        pltpu.VMEM((CHG,), i32),
        pltpu.VMEM((CHG,), i32),
        pltpu.VMEM((CHG, H), f32),
        pltpu.VMEM((CHG, H), f32),
        pltpu.VMEM((CHG, 16), f32),
        pltpu.VMEM((CHG, 16), f32),
        pltpu.VMEM_SHARED((NP, H), f32),
        pltpu.SemaphoreType.DMA,
        pltpu.SemaphoreType.DMA,
    ],
)
def _sgat(hw_hbm, ee_hbm, sp_hbm, dp_hbm, zeros_hbm, out_hbm,
          sidx_a, sidx_b, didx_a, didx_b, rows_a, rows_b, ee_a, ee_b,
          acc_sh, sem_a, sem_b):
    cid, sid, wid = _tile_ids()
    pltpu.sync_copy(zeros_hbm.at[pl.ds(sid * STRIPE, STRIPE)],
                    acc_sh.at[pl.ds(sid * STRIPE, STRIPE)])
    plsc.subcore_barrier()
    base = jnp.where(cid == 0, sid * KF_G, NS * KF_G + sid * KS_G) * CHG
    nch = jnp.where(cid == 0, KF_G, KS_G)
    pltpu.sync_copy(sp_hbm.at[pl.ds(base, CHG)], sidx_a)
    pltpu.sync_copy(dp_hbm.at[pl.ds(base, CHG)], didx_a)
    pltpu.async_copy(hw_hbm.at[sidx_a], rows_a, sem_a)
    pltpu.sync_copy(ee_hbm.at[pl.ds(base, CHG)], ee_a)

    def compute(rows_v, ee_v):
        def inner(i, cc):
            ee = ee_v[i, :]
            for h in range(HEADS):
                c_h = ee[h]
                rows_v[i, pl.ds(32 * h, 16)] = (
                    rows_v[i, pl.ds(32 * h, 16)] * c_h)
                rows_v[i, pl.ds(32 * h + 16, 16)] = (
                    rows_v[i, pl.ds(32 * h + 16, 16)] * c_h)
            return cc

        lax.fori_loop(0, CHG, inner, 0)

    def step(j2, c):
        off = base + 2 * j2 * CHG
        pltpu.sync_copy(sp_hbm.at[pl.ds(off + CHG, CHG)], sidx_b)
        pltpu.sync_copy(dp_hbm.at[pl.ds(off + CHG, CHG)], didx_b)
        pltpu.sync_copy(ee_hbm.at[pl.ds(off + CHG, CHG)], ee_b)
        pltpu.make_async_copy(hw_hbm.at[sidx_a], rows_a, sem_a).wait()
        pltpu.async_copy(hw_hbm.at[sidx_b], rows_b, sem_b)
        compute(rows_a, ee_a)
        pltpu.sync_copy(rows_a, acc_sh.at[didx_a], add=True)

        @pl.when(2 * j2 + 2 < nch)
        def _():
            pltpu.sync_copy(sp_hbm.at[pl.ds(off + 2 * CHG, CHG)], sidx_a)
            pltpu.sync_copy(dp_hbm.at[pl.ds(off + 2 * CHG, CHG)], didx_a)
            pltpu.sync_copy(ee_hbm.at[pl.ds(off + 2 * CHG, CHG)], ee_a)

        pltpu.make_async_copy(hw_hbm.at[sidx_b], rows_b, sem_b).wait()

        @pl.when(2 * j2 + 2 < nch)
        def _():
            pltpu.async_copy(hw_hbm.at[sidx_a], rows_a, sem_a)

        compute(rows_b, ee_b)
        pltpu.sync_copy(rows_b, acc_sh.at[didx_b], add=True)
        return c

    lax.fori_loop(0, nch // 2, step, 0)
    plsc.subcore_barrier()
    pltpu.sync_copy(acc_sh.at[pl.ds(sid * STRIPE, STRIPE)],
                    out_hbm.at[cid, pl.ds(sid * STRIPE, STRIPE)])


def _k2_body(x_ref, w_ref, degs_ref, q_ref, dinv_ref):
    degs = degs_ref[...]
    deg = degs[0, :, :1] + degs[1, :, :1] + 1.0
    dinv = lax.rsqrt(deg)
    dinv_ref[...] = dinv
    q_ref[...] = dinv * jnp.dot(x_ref[...], w_ref[...],
                                preferred_element_type=f32)


_k2 = pl.pallas_call(
    _k2_body,
    grid=(NBLK,),
    in_specs=[
        pl.BlockSpec((BLK, H), lambda i: (i, 0)),
        pl.BlockSpec((H, H), lambda i: (0, 0)),
        pl.BlockSpec((NC, BLK, 16), lambda i: (0, i, 0)),
    ],
    out_specs=[
        pl.BlockSpec((BLK, H), lambda i: (i, 0)),
        pl.BlockSpec((BLK, 1), lambda i: (i, 0)),
    ],
    out_shape=[
        jax.ShapeDtypeStruct((N, H), f32),
        jax.ShapeDtypeStruct((N, 1), f32),
    ],
)


def _ka_body(acc_ref, q_ref, res_ref, dinv_ref, b_ref, w_ref, h_ref, qn_ref):
    acc = acc_ref[...]
    dinv = dinv_ref[...]
    y = dinv * (acc[0] + acc[1] + q_ref[...]) + b_ref[...]
    h = res_ref[...] + jnp.maximum(y, 0.0)
    h_ref[...] = h
    qn_ref[...] = dinv * jnp.dot(h, w_ref[...], preferred_element_type=f32)


_ka = pl.pallas_call(
    _ka_body,
    grid=(NBLK,),
    in_specs=[
        pl.BlockSpec((NC, BLK, H), lambda i: (0, i, 0)),
        pl.BlockSpec((BLK, H), lambda i: (i, 0)),
        pl.BlockSpec((BLK, H), lambda i: (i, 0)),
        pl.BlockSpec((BLK, 1), lambda i: (i, 0)),
        pl.BlockSpec((1, H), lambda i: (0, 0)),
        pl.BlockSpec((H, H), lambda i: (0, 0)),
    ],
    out_specs=[
        pl.BlockSpec((BLK, H), lambda i: (i, 0)),
        pl.BlockSpec((BLK, H), lambda i: (i, 0)),
    ],
    out_shape=[
        jax.ShapeDtypeStruct((N, H), f32),
        jax.ShapeDtypeStruct((N, H), f32),
    ],
)


def _kb_body(acc_ref, q_ref, res_ref, dinv_ref, b_ref, wg_ref, atts_ref,
             attd_ref, hw_ref, as_ref, ad_ref, m_ref, ms_acc, md_acc):
    i = pl.program_id(0)
    acc = acc_ref[...]
    dinv = dinv_ref[...]
    y = dinv * (acc[0] + acc[1] + q_ref[...]) + b_ref[...]
    h = res_ref[...] + jnp.maximum(y, 0.0)
    hw = jnp.dot(h, wg_ref[...], preferred_element_type=f32)
    hw_ref[...] = hw
    lane = lax.broadcasted_iota(i32, (H, 16), 0)
    col = lax.broadcasted_iota(i32, (H, 16), 1)
    e16 = jnp.where((col < HEADS) & (lane // DH == col), 1.0, 0.0)
    as16 = jnp.dot(hw * atts_ref[...], e16, preferred_element_type=f32)
    ad16 = jnp.dot(hw * attd_ref[...], e16, preferred_element_type=f32)
    as_ref[...] = as16
    ad_ref[...] = ad16
    pms = jnp.max(as16, axis=0, keepdims=True)
    pmd = jnp.max(ad16, axis=0, keepdims=True)

    @pl.when(i == 0)
    def _():
        ms_acc[...] = pms
        md_acc[...] = pmd

    @pl.when(i > 0)
    def _():
        ms_acc[...] = jnp.maximum(ms_acc[...], pms)
        md_acc[...] = jnp.maximum(md_acc[...], pmd)

    @pl.when(i == NBLK - 1)
    def _():
        m_ref[...] = ms_acc[...] + md_acc[...]


_kb = pl.pallas_call(
    _kb_body,
    grid=(NBLK,),
    in_specs=[
        pl.BlockSpec((NC, BLK, H), lambda i: (0, i, 0)),
        pl.BlockSpec((BLK, H), lambda i: (i, 0)),
        pl.BlockSpec((BLK, H), lambda i: (i, 0)),
        pl.BlockSpec((BLK, 1), lambda i: (i, 0)),
        pl.BlockSpec((1, H), lambda i: (0, 0)),
        pl.BlockSpec((H, H), lambda i: (0, 0)),
        pl.BlockSpec((1, H), lambda i: (0, 0)),
        pl.BlockSpec((1, H), lambda i: (0, 0)),
    ],
    out_specs=[
        pl.BlockSpec((BLK, H), lambda i: (i, 0)),
        pl.BlockSpec((BLK, 16), lambda i: (i, 0)),
        pl.BlockSpec((BLK, 16), lambda i: (i, 0)),
        pl.BlockSpec((1, 16), lambda i: (0, 0)),
    ],
    out_shape=[
        jax.ShapeDtypeStruct((N, H), f32),
        jax.ShapeDtypeStruct((N, 16), f32),
        jax.ShapeDtypeStruct((N, 16), f32),
        jax.ShapeDtypeStruct((1, 16), f32),
    ],
    scratch_shapes=[
        pltpu.VMEM((1, 16), f32),
        pltpu.VMEM((1, 16), f32),
    ],
)


def _k7_body(gacc_ref, hw_ref, as_ref, ad_ref, m_ref, dens_ref, bg_ref,
             bcol_ref, p1w_ref, p1b_ref, p2w_ref, p2b_ref,
             p3w_ref, p3b_ref, out_ref, gm_acc, gx_acc, cnt_acc):
    i = pl.program_id(0)

    @pl.when(i == 0)
    def _():
        gm_acc[...] = jnp.zeros((G, H), f32)
        gx_acc[...] = jnp.full((G, H), -jnp.inf, f32)
        cnt_acc[...] = jnp.zeros((G, 1), f32)

    a = as_ref[...] + ad_ref[...]
    ee_self = jnp.exp(jnp.maximum(a, 0.2 * a) - m_ref[...])
    dens = dens_ref[...]
    den16 = dens[0] + dens[1] + ee_self
    colc = lax.broadcasted_iota(i32, (16, H), 0)
    lanec = lax.broadcasted_iota(i32, (16, H), 1)
    t16 = jnp.where((colc < HEADS) & (lanec // DH == colc), 1.0, 0.0)
    den128 = jnp.dot(den16, t16, preferred_element_type=f32)
    ee128 = jnp.dot(ee_self, t16, preferred_element_type=f32)
    gacc = gacc_ref[...]
    g = (gacc[0] + gacc[1] + ee128 * hw_ref[...]) / den128 + bg_ref[...]

    bcol = bcol_ref[...]
    grow = lax.broadcasted_iota(i32, (1, G), 1)
    onehot = jnp.where(bcol == grow, 1.0, 0.0)
    dn = (((0,), (0,)), ((), ()))
    gm_acc[...] = gm_acc[...] + lax.dot_general(
        onehot, g, dn, preferred_element_type=f32)
    cnt_acc[...] = cnt_acc[...] + lax.dot_general(
        onehot, jnp.ones((BLK, 1), f32), dn, preferred_element_type=f32)

    g0 = bcol_ref[0, 0]
    g1 = bcol_ref[BLK - 1, 0]

    def gbody(gi, c):
        vals = jnp.where(bcol == gi, g, -jnp.inf)
        m = jnp.max(vals, axis=0, keepdims=True)
        cur = gx_acc[pl.ds(gi, 1), :]
        gx_acc[pl.ds(gi, 1), :] = jnp.maximum(cur, m)
        return c

    lax.fori_loop(g0, g1 + 1, gbody, 0)

    @pl.when(i == NBLK - 1)
    def _():
        cnt = jnp.maximum(cnt_acc[...], 1.0)
        z = jnp.concatenate([gm_acc[...] / cnt, gx_acc[...]], axis=1)
        z = jnp.maximum(jnp.dot(z, p1w_ref[...], preferred_element_type=f32)
                        + p1b_ref[...], 0.0)
        z = jnp.maximum(jnp.dot(z, p2w_ref[...], preferred_element_type=f32)
                        + p2b_ref[...], 0.0)
        out_ref[...] = (jnp.dot(z, p3w_ref[...], preferred_element_type=f32)
                        + p3b_ref[...])


_k7 = pl.pallas_call(
    _k7_body,
    grid=(NBLK,),
    in_specs=[
        pl.BlockSpec((NC, BLK, H), lambda i: (0, i, 0)),
        pl.BlockSpec((BLK, H), lambda i: (i, 0)),
        pl.BlockSpec((BLK, 16), lambda i: (i, 0)),
        pl.BlockSpec((BLK, 16), lambda i: (i, 0)),
        pl.BlockSpec((1, 16), lambda i: (0, 0)),
        pl.BlockSpec((NC, BLK, 16), lambda i: (0, i, 0)),
        pl.BlockSpec((1, H), lambda i: (0, 0)),
        pl.BlockSpec((BLK, 1), lambda i: (i, 0)),
        pl.BlockSpec((2 * H, H // 2), lambda i: (0, 0)),
        pl.BlockSpec((1, H // 2), lambda i: (0, 0)),
        pl.BlockSpec((H // 2, H // 4), lambda i: (0, 0)),
        pl.BlockSpec((1, H // 4), lambda i: (0, 0)),
        pl.BlockSpec((H // 4, 1), lambda i: (0, 0)),
        pl.BlockSpec((1, 1), lambda i: (0, 0)),
    ],
    out_specs=pl.BlockSpec((G, 1), lambda i: (0, 0)),
    out_shape=jax.ShapeDtypeStruct((G, 1), f32),
    scratch_shapes=[
        pltpu.VMEM((G, H), f32),
        pltpu.VMEM((G, H), f32),
        pltpu.VMEM((G, 1), f32),
    ],
)


def kernel(x, edge_index, batch, W1, b1, W2, b2, W3, b3, Wg, att_src,
           att_dst, bg, P1w, P1b, P2w, P2b, P3w, P3b):
    s = edge_index[0]
    d = edge_index[1]
    pad = EP - E
    sp = jnp.concatenate([s, jnp.zeros((pad,), i32)])
    dp = jnp.concatenate([d, jnp.full((pad,), N, i32)])
    zeros128 = jnp.zeros((NP, H), f32)
    zeros16 = jnp.zeros((NP, 16), f32)
    ones16 = jnp.ones((CH, 16), f32)

    degs = _sdeg(dp, ones16, zeros16)
    q1, dinv = _k2(x, W1, degs)
    acc1 = _sscat(q1, sp, dp, zeros128)
    zres = jnp.zeros((N, H), f32)
    h1, q2 = _ka(acc1, q1, zres, dinv, b1.reshape(1, H), W2)
    acc2 = _sscat(q2, sp, dp, zeros128)
    h2, q3 = _ka(acc2, q2, h1, dinv, b2.reshape(1, H), W3)
    acc3 = _sscat(q3, sp, dp, zeros128)
    hw, as16, ad16, m16 = _kb(acc3, q3, h2, dinv, b3.reshape(1, H), Wg,
                              att_src.reshape(1, H), att_dst.reshape(1, H))
    ee, dens = _satt(as16, ad16, m16, sp, dp, zeros16)
    gacc = _sgat(hw, ee, sp, dp, zeros128)
    out = _k7(gacc, hw, as16, ad16, m16, dens, bg.reshape(1, H),
              batch.reshape(N, 1), P1w,
              P1b.reshape(1, H // 2), P2w, P2b.reshape(1, H // 4), P3w,
              P3b.reshape(1, 1))
    return out.reshape(-1)


# speedup vs baseline: 1.2409x; 1.2409x over previous
import functools

import jax
import jax.numpy as jnp
from jax import lax
from jax.experimental import pallas as pl
from jax.experimental.pallas import tpu as pltpu
from jax.experimental.pallas import tpu_sc as plsc

N = 10000
E = 320000
H = 128
G = 64
HEADS = 4
DH = 32

NC = 2
NS = 16
NW = NC * NS
CH = 128
PTE = 10240
NCHUNK = PTE // CH
EP = NW * PTE
NP = 10112
STRIPE = NP // NS
KF_S, KS_S = 128, 32
KF_G, KS_G = 178, 142
NBLK = 10
BLK = N // NBLK

f32 = jnp.float32
i32 = jnp.int32

_mesh = plsc.VectorSubcoreMesh(core_axis_name="c", subcore_axis_name="s")
_untiled = pltpu.CompilerParams(use_tc_tiling_on_sc=False)


def _tile_ids():
    cid = lax.axis_index("c")
    sid = lax.axis_index("s")
    return cid, sid, cid * NS + sid


@functools.partial(
    pl.kernel,
    out_type=jax.ShapeDtypeStruct((NC, NP, 16), f32),
    mesh=_mesh,
    scratch_types=[
        pltpu.VMEM((CH,), i32),
        pltpu.VMEM((CH, 16), f32),
        pltpu.VMEM_SHARED((NP, 16), f32),
        pltpu.SemaphoreType.DMA,
    ],
)
def _sdeg(dp_hbm, ones_hbm, zeros_hbm, out_hbm, didx_v, ones_v, acc_sh, sem):
    cid, sid, wid = _tile_ids()
    pltpu.sync_copy(zeros_hbm.at[pl.ds(sid * STRIPE, STRIPE)],
                    acc_sh.at[pl.ds(sid * STRIPE, STRIPE)])
    pltpu.sync_copy(ones_hbm, ones_v)
    plsc.subcore_barrier()
    base = wid * PTE

    def step(j, c):
        pltpu.sync_copy(dp_hbm.at[pl.ds(base + j * CH, CH)], didx_v)
        pltpu.sync_copy(ones_v, acc_sh.at[didx_v], add=True)
        return c

    lax.fori_loop(0, NCHUNK, step, 0)
    plsc.subcore_barrier()
    pltpu.sync_copy(acc_sh.at[pl.ds(sid * STRIPE, STRIPE)],
                    out_hbm.at[cid, pl.ds(sid * STRIPE, STRIPE)])


@functools.partial(
    pl.kernel,
    out_type=jax.ShapeDtypeStruct((NC, NP, H), f32),
    mesh=_mesh,
    scratch_types=[
        pltpu.VMEM((CH,), i32),
        pltpu.VMEM((CH,), i32),
        pltpu.VMEM((CH,), i32),
        pltpu.VMEM((CH,), i32),
        pltpu.VMEM((CH, H), f32),
        pltpu.VMEM((CH, H), f32),
        pltpu.VMEM_SHARED((NP, H), f32),
        pltpu.SemaphoreType.DMA,
        pltpu.SemaphoreType.DMA,
    ],
)
def _sscat(q_hbm, sp_hbm, dp_hbm, zeros_hbm, out_hbm,
           sidx_a, sidx_b, didx_a, didx_b, rows_a, rows_b,
           acc_sh, sem_a, sem_b):
    cid, sid, wid = _tile_ids()
    pltpu.sync_copy(zeros_hbm.at[pl.ds(sid * STRIPE, STRIPE)],
                    acc_sh.at[pl.ds(sid * STRIPE, STRIPE)])
    plsc.subcore_barrier()
    base = jnp.where(cid == 0, sid * KF_S, NS * KF_S + sid * KS_S) * CH
    nch = jnp.where(cid == 0, KF_S, KS_S)

    @pl.when(nch > 0)
    def _():
        pltpu.sync_copy(sp_hbm.at[pl.ds(base, CH)], sidx_a)
        pltpu.sync_copy(dp_hbm.at[pl.ds(base, CH)], didx_a)
        pltpu.async_copy(q_hbm.at[sidx_a], rows_a, sem_a)

    def step(j2, c):
        o = base + 2 * j2 * CH
        pltpu.sync_copy(sp_hbm.at[pl.ds(o + CH, CH)], sidx_b)
        pltpu.sync_copy(dp_hbm.at[pl.ds(o + CH, CH)], didx_b)
        pltpu.make_async_copy(q_hbm.at[sidx_a], rows_a, sem_a).wait()
        pltpu.async_copy(q_hbm.at[sidx_b], rows_b, sem_b)
        pltpu.sync_copy(rows_a, acc_sh.at[didx_a], add=True)

        @pl.when(2 * j2 + 2 < nch)
        def _():
            pltpu.sync_copy(sp_hbm.at[pl.ds(o + 2 * CH, CH)], sidx_a)
            pltpu.sync_copy(dp_hbm.at[pl.ds(o + 2 * CH, CH)], didx_a)

        pltpu.make_async_copy(q_hbm.at[sidx_b], rows_b, sem_b).wait()

        @pl.when(2 * j2 + 2 < nch)
        def _():
            pltpu.async_copy(q_hbm.at[sidx_a], rows_a, sem_a)

        pltpu.sync_copy(rows_b, acc_sh.at[didx_b], add=True)
        return c

    lax.fori_loop(0, nch // 2, step, 0)
    plsc.subcore_barrier()
    pltpu.sync_copy(acc_sh.at[pl.ds(sid * STRIPE, STRIPE)],
                    out_hbm.at[cid, pl.ds(sid * STRIPE, STRIPE)])


@functools.partial(
    pl.kernel,
    out_type=(jax.ShapeDtypeStruct((EP, 16), f32),
              jax.ShapeDtypeStruct((NC, NP, 16), f32)),
    mesh=_mesh,
    compiler_params=_untiled,
    scratch_types=[
        pltpu.VMEM((CH,), i32),
        pltpu.VMEM((CH,), i32),
        pltpu.VMEM((CH,), i32),
        pltpu.VMEM((CH,), i32),
        pltpu.VMEM((CH, 16), f32),
        pltpu.VMEM((CH, 16), f32),
        pltpu.VMEM((CH, 16), f32),
        pltpu.VMEM((CH, 16), f32),
        pltpu.VMEM((CH, 16), f32),
        pltpu.VMEM((1, 16), f32),
        pltpu.VMEM_SHARED((NP, 16), f32),
        pltpu.SemaphoreType.DMA,
        pltpu.SemaphoreType.DMA,
    ],
)
def _satt(as_hbm, ad_hbm, m_hbm, sp_hbm, dp_hbm, zeros_hbm,
          ee_hbm, dens_hbm,
          sidx_a, sidx_b, didx_a, didx_b, as_a, as_b, ad_a, ad_b,
          ee_v, m_v, den_sh, sem_a, sem_b):
    cid, sid, wid = _tile_ids()
    pltpu.sync_copy(zeros_hbm.at[pl.ds(sid * STRIPE, STRIPE)],
                    den_sh.at[pl.ds(sid * STRIPE, STRIPE)])
    pltpu.sync_copy(m_hbm, m_v)
    plsc.subcore_barrier()
    base = wid * PTE
    pltpu.sync_copy(sp_hbm.at[pl.ds(base, CH)], sidx_a)
    pltpu.sync_copy(dp_hbm.at[pl.ds(base, CH)], didx_a)
    pltpu.async_copy(as_hbm.at[sidx_a], as_a, sem_a)
    pltpu.async_copy(ad_hbm.at[didx_a], ad_a, sem_a)

    def compute(as_v, ad_v):
        def inner(i, cc):
            m16 = m_v[0, :]
            a = as_v[i, :] + ad_v[i, :]
            ee_v[i, :] = jnp.exp(jnp.maximum(a, 0.2 * a) - m16)
            return cc

        lax.fori_loop(0, CH, inner, 0)

    def step(j2, c):
        off = base + 2 * j2 * CH
        pltpu.sync_copy(sp_hbm.at[pl.ds(off + CH, CH)], sidx_b)
        pltpu.sync_copy(dp_hbm.at[pl.ds(off + CH, CH)], didx_b)
        pltpu.make_async_copy(as_hbm.at[sidx_a], as_a, sem_a).wait()
        pltpu.make_async_copy(ad_hbm.at[didx_a], ad_a, sem_a).wait()
        pltpu.async_copy(as_hbm.at[sidx_b], as_b, sem_b)
        pltpu.async_copy(ad_hbm.at[didx_b], ad_b, sem_b)
        compute(as_a, ad_a)
        pltpu.sync_copy(ee_v, ee_hbm.at[pl.ds(off, CH)])
        pltpu.sync_copy(ee_v, den_sh.at[didx_a], add=True)

        @pl.when(2 * j2 + 2 < NCHUNK)
        def _():
            pltpu.sync_copy(sp_hbm.at[pl.ds(off + 2 * CH, CH)], sidx_a)
            pltpu.sync_copy(dp_hbm.at[pl.ds(off + 2 * CH, CH)], didx_a)

        pltpu.make_async_copy(as_hbm.at[sidx_b], as_b, sem_b).wait()
        pltpu.make_async_copy(ad_hbm.at[didx_b], ad_b, sem_b).wait()

        @pl.when(2 * j2 + 2 < NCHUNK)
        def _():
            pltpu.async_copy(as_hbm.at[sidx_a], as_a, sem_a)
            pltpu.async_copy(ad_hbm.at[didx_a], ad_a, sem_a)

        compute(as_b, ad_b)
        pltpu.sync_copy(ee_v, ee_hbm.at[pl.ds(off + CH, CH)])
        pltpu.sync_copy(ee_v, den_sh.at[didx_b], add=True)
        return c

    lax.fori_loop(0, NCHUNK // 2, step, 0)
    plsc.subcore_barrier()
    pltpu.sync_copy(den_sh.at[pl.ds(sid * STRIPE, STRIPE)],
                    dens_hbm.at[cid, pl.ds(sid * STRIPE, STRIPE)])


CHG = 64
NCHG = PTE // CHG


@functools.partial(
    pl.kernel,
    out_type=jax.ShapeDtypeStruct((NC, NP, H), f32),
    mesh=_mesh,
    scratch_types=[
        pltpu.VMEM((CHG,), i32),
        pltpu.VMEM((CHG,), i32),
        pltpu.VMEM((CHG,), i32),
        pltpu.VMEM((CHG,), i32),
        pltpu.VMEM((CHG, H), f32),
        pltpu.VMEM((CHG, H), f32),
        pltpu.VMEM((CHG, 16), f32),
        pltpu.VMEM((CHG, 16), f32),
        pltpu.VMEM_SHARED((NP, H), f32),
        pltpu.SemaphoreType.DMA,
        pltpu.SemaphoreType.DMA,
    ],
)
def _sgat(hw_hbm, ee_hbm, sp_hbm, dp_hbm, zeros_hbm, out_hbm,
          sidx_a, sidx_b, didx_a, didx_b, rows_a, rows_b, ee_a, ee_b,
          acc_sh, sem_a, sem_b):
    cid, sid, wid = _tile_ids()
    pltpu.sync_copy(zeros_hbm.at[pl.ds(sid * STRIPE, STRIPE)],
                    acc_sh.at[pl.ds(sid * STRIPE, STRIPE)])
    plsc.subcore_barrier()
    base = jnp.where(cid == 0, sid * KF_G, NS * KF_G + sid * KS_G) * CHG
    nch = jnp.where(cid == 0, KF_G, KS_G)
    pltpu.sync_copy(sp_hbm.at[pl.ds(base, CHG)], sidx_a)
    pltpu.sync_copy(dp_hbm.at[pl.ds(base, CHG)], didx_a)
    pltpu.async_copy(hw_hbm.at[sidx_a], rows_a, sem_a)
    pltpu.sync_copy(ee_hbm.at[pl.ds(base, CHG)], ee_a)

    def compute(rows_v, ee_v):
        def inner(i, cc):
            ee = ee_v[i, :]
            for h in range(HEADS):
                c_h = ee[h]
                rows_v[i, pl.ds(32 * h, 16)] = (
                    rows_v[i, pl.ds(32 * h, 16)] * c_h)
                rows_v[i, pl.ds(32 * h + 16, 16)] = (
                    rows_v[i, pl.ds(32 * h + 16, 16)] * c_h)
            return cc

        lax.fori_loop(0, CHG, inner, 0)

    def step(j2, c):
        off = base + 2 * j2 * CHG
        pltpu.sync_copy(sp_hbm.at[pl.ds(off + CHG, CHG)], sidx_b)
        pltpu.sync_copy(dp_hbm.at[pl.ds(off + CHG, CHG)], didx_b)
        pltpu.sync_copy(ee_hbm.at[pl.ds(off + CHG, CHG)], ee_b)
        pltpu.make_async_copy(hw_hbm.at[sidx_a], rows_a, sem_a).wait()
        pltpu.async_copy(hw_hbm.at[sidx_b], rows_b, sem_b)
        compute(rows_a, ee_a)
        pltpu.sync_copy(rows_a, acc_sh.at[didx_a], add=True)

        @pl.when(2 * j2 + 2 < nch)
        def _():
            pltpu.sync_copy(sp_hbm.at[pl.ds(off + 2 * CHG, CHG)], sidx_a)
            pltpu.sync_copy(dp_hbm.at[pl.ds(off + 2 * CHG, CHG)], didx_a)
            pltpu.sync_copy(ee_hbm.at[pl.ds(off + 2 * CHG, CHG)], ee_a)

        pltpu.make_async_copy(hw_hbm.at[sidx_b], rows_b, sem_b).wait()

        @pl.when(2 * j2 + 2 < nch)
        def _():
            pltpu.async_copy(hw_hbm.at[sidx_a], rows_a, sem_a)

        compute(rows_b, ee_b)
        pltpu.sync_copy(rows_b, acc_sh.at[didx_b], add=True)
        return c

    lax.fori_loop(0, nch // 2, step, 0)
    plsc.subcore_barrier()
    pltpu.sync_copy(acc_sh.at[pl.ds(sid * STRIPE, STRIPE)],
                    out_hbm.at[cid, pl.ds(sid * STRIPE, STRIPE)])


def _k2_body(x_ref, w_ref, degs_ref, q_ref, dinv_ref):
    degs = degs_ref[...]
    deg = degs[0, :, :1] + degs[1, :, :1] + 1.0
    dinv = lax.rsqrt(deg)
    dinv_ref[...] = dinv
    q_ref[...] = dinv * jnp.dot(x_ref[...], w_ref[...],
                                preferred_element_type=f32)


_k2 = pl.pallas_call(
    _k2_body,
    grid=(NBLK,),
    in_specs=[
        pl.BlockSpec((BLK, H), lambda i: (i, 0)),
        pl.BlockSpec((H, H), lambda i: (0, 0)),
        pl.BlockSpec((NC, BLK, 16), lambda i: (0, i, 0)),
    ],
    out_specs=[
        pl.BlockSpec((BLK, H), lambda i: (i, 0)),
        pl.BlockSpec((BLK, 1), lambda i: (i, 0)),
    ],
    out_shape=[
        jax.ShapeDtypeStruct((N, H), f32),
        jax.ShapeDtypeStruct((N, 1), f32),
    ],
)


def _ka_body(acc_ref, q_ref, res_ref, dinv_ref, b_ref, w_ref, h_ref, qn_ref):
    acc = acc_ref[...]
    dinv = dinv_ref[...]
    y = dinv * (acc[0] + acc[1] + q_ref[...]) + b_ref[...]
    h = res_ref[...] + jnp.maximum(y, 0.0)
    h_ref[...] = h
    qn_ref[...] = dinv * jnp.dot(h, w_ref[...], preferred_element_type=f32)


_ka = pl.pallas_call(
    _ka_body,
    grid=(NBLK,),
    in_specs=[
        pl.BlockSpec((NC, BLK, H), lambda i: (0, i, 0)),
        pl.BlockSpec((BLK, H), lambda i: (i, 0)),
        pl.BlockSpec((BLK, H), lambda i: (i, 0)),
        pl.BlockSpec((BLK, 1), lambda i: (i, 0)),
        pl.BlockSpec((1, H), lambda i: (0, 0)),
        pl.BlockSpec((H, H), lambda i: (0, 0)),
    ],
    out_specs=[
        pl.BlockSpec((BLK, H), lambda i: (i, 0)),
        pl.BlockSpec((BLK, H), lambda i: (i, 0)),
    ],
    out_shape=[
        jax.ShapeDtypeStruct((N, H), f32),
        jax.ShapeDtypeStruct((N, H), f32),
    ],
)


def _kb_body(acc_ref, q_ref, res_ref, dinv_ref, b_ref, wg_ref, atts_ref,
             attd_ref, hw_ref, as_ref, ad_ref, m_ref, ms_acc, md_acc):
    i = pl.program_id(0)
    acc = acc_ref[...]
    dinv = dinv_ref[...]
    y = dinv * (acc[0] + acc[1] + q_ref[...]) + b_ref[...]
    h = res_ref[...] + jnp.maximum(y, 0.0)
    hw = jnp.dot(h, wg_ref[...], preferred_element_type=f32)
    hw_ref[...] = hw
    lane = lax.broadcasted_iota(i32, (H, 16), 0)
    col = lax.broadcasted_iota(i32, (H, 16), 1)
    e16 = jnp.where((col < HEADS) & (lane // DH == col), 1.0, 0.0)
    as16 = jnp.dot(hw * atts_ref[...], e16, preferred_element_type=f32)
    ad16 = jnp.dot(hw * attd_ref[...], e16, preferred_element_type=f32)
    as_ref[...] = as16
    ad_ref[...] = ad16
    pms = jnp.max(as16, axis=0, keepdims=True)
    pmd = jnp.max(ad16, axis=0, keepdims=True)

    @pl.when(i == 0)
    def _():
        ms_acc[...] = pms
        md_acc[...] = pmd

    @pl.when(i > 0)
    def _():
        ms_acc[...] = jnp.maximum(ms_acc[...], pms)
        md_acc[...] = jnp.maximum(md_acc[...], pmd)

    @pl.when(i == NBLK - 1)
    def _():
        m_ref[...] = ms_acc[...] + md_acc[...]


_kb = pl.pallas_call(
    _kb_body,
    grid=(NBLK,),
    in_specs=[
        pl.BlockSpec((NC, BLK, H), lambda i: (0, i, 0)),
        pl.BlockSpec((BLK, H), lambda i: (i, 0)),
        pl.BlockSpec((BLK, H), lambda i: (i, 0)),
        pl.BlockSpec((BLK, 1), lambda i: (i, 0)),
        pl.BlockSpec((1, H), lambda i: (0, 0)),
        pl.BlockSpec((H, H), lambda i: (0, 0)),
        pl.BlockSpec((1, H), lambda i: (0, 0)),
        pl.BlockSpec((1, H), lambda i: (0, 0)),
    ],
    out_specs=[
        pl.BlockSpec((BLK, H), lambda i: (i, 0)),
        pl.BlockSpec((BLK, 16), lambda i: (i, 0)),
        pl.BlockSpec((BLK, 16), lambda i: (i, 0)),
        pl.BlockSpec((1, 16), lambda i: (0, 0)),
    ],
    out_shape=[
        jax.ShapeDtypeStruct((N, H), f32),
        jax.ShapeDtypeStruct((N, 16), f32),
        jax.ShapeDtypeStruct((N, 16), f32),
        jax.ShapeDtypeStruct((1, 16), f32),
    ],
    scratch_shapes=[
        pltpu.VMEM((1, 16), f32),
        pltpu.VMEM((1, 16), f32),
    ],
)


def _k7_body(gacc_ref, hw_ref, as_ref, ad_ref, m_ref, dens_ref, bg_ref,
             bcol_ref, p1w_ref, p1b_ref, p2w_ref, p2b_ref,
             p3w_ref, p3b_ref, out_ref, gm_acc, gx_acc, cnt_acc):
    i = pl.program_id(0)

    @pl.when(i == 0)
    def _():
        gm_acc[...] = jnp.zeros((G, H), f32)
        gx_acc[...] = jnp.full((G, H), -jnp.inf, f32)
        cnt_acc[...] = jnp.zeros((G, 1), f32)

    a = as_ref[...] + ad_ref[...]
    ee_self = jnp.exp(jnp.maximum(a, 0.2 * a) - m_ref[...])
    dens = dens_ref[...]
    den16 = dens[0] + dens[1] + ee_self
    colc = lax.broadcasted_iota(i32, (16, H), 0)
    lanec = lax.broadcasted_iota(i32, (16, H), 1)
    t16 = jnp.where((colc < HEADS) & (lanec // DH == colc), 1.0, 0.0)
    den128 = jnp.dot(den16, t16, preferred_element_type=f32)
    ee128 = jnp.dot(ee_self, t16, preferred_element_type=f32)
    gacc = gacc_ref[...]
    g = (gacc[0] + gacc[1] + ee128 * hw_ref[...]) / den128 + bg_ref[...]

    bcol = bcol_ref[...]
    grow = lax.broadcasted_iota(i32, (1, G), 1)
    onehot = jnp.where(bcol == grow, 1.0, 0.0)
    dn = (((0,), (0,)), ((), ()))
    gm_acc[...] = gm_acc[...] + lax.dot_general(
        onehot, g, dn, preferred_element_type=f32)
    cnt_acc[...] = cnt_acc[...] + lax.dot_general(
        onehot, jnp.ones((BLK, 1), f32), dn, preferred_element_type=f32)

    g0 = bcol_ref[0, 0]
    g1 = bcol_ref[BLK - 1, 0]

    def gbody(gi, c):
        vals = jnp.where(bcol == gi, g, -jnp.inf)
        m = jnp.max(vals, axis=0, keepdims=True)
        cur = gx_acc[pl.ds(gi, 1), :]
        gx_acc[pl.ds(gi, 1), :] = jnp.maximum(cur, m)
        return c

    lax.fori_loop(g0, g1 + 1, gbody, 0)

    @pl.when(i == NBLK - 1)
    def _():
        cnt = jnp.maximum(cnt_acc[...], 1.0)
        z = jnp.concatenate([gm_acc[...] / cnt, gx_acc[...]], axis=1)
        z = jnp.maximum(jnp.dot(z, p1w_ref[...], preferred_element_type=f32)
                        + p1b_ref[...], 0.0)
        z = jnp.maximum(jnp.dot(z, p2w_ref[...], preferred_element_type=f32)
                        + p2b_ref[...], 0.0)
        out_ref[...] = (jnp.dot(z, p3w_ref[...], preferred_element_type=f32)
                        + p3b_ref[...])


_k7 = pl.pallas_call(
    _k7_body,
    grid=(NBLK,),
    in_specs=[
        pl.BlockSpec((NC, BLK, H), lambda i: (0, i, 0)),
        pl.BlockSpec((BLK, H), lambda i: (i, 0)),
        pl.BlockSpec((BLK, 16), lambda i: (i, 0)),
        pl.BlockSpec((BLK, 16), lambda i: (i, 0)),
        pl.BlockSpec((1, 16), lambda i: (0, 0)),
        pl.BlockSpec((NC, BLK, 16), lambda i: (0, i, 0)),
        pl.BlockSpec((1, H), lambda i: (0, 0)),
        pl.BlockSpec((BLK, 1), lambda i: (i, 0)),
        pl.BlockSpec((2 * H, H // 2), lambda i: (0, 0)),
        pl.BlockSpec((1, H // 2), lambda i: (0, 0)),
        pl.BlockSpec((H // 2, H // 4), lambda i: (0, 0)),
        pl.BlockSpec((1, H // 4), lambda i: (0, 0)),
        pl.BlockSpec((H // 4, 1), lambda i: (0, 0)),
        pl.BlockSpec((1, 1), lambda i: (0, 0)),
    ],
    out_specs=pl.BlockSpec((G, 1), lambda i: (0, 0)),
    out_shape=jax.ShapeDtypeStruct((G, 1), f32),
    scratch_shapes=[
        pltpu.VMEM((G, H), f32),
        pltpu.VMEM((G, H), f32),
        pltpu.VMEM((G, 1), f32),
    ],
)


def kernel(x, edge_index, batch, W1, b1, W2, b2, W3, b3, Wg, att_src,
           att_dst, bg, P1w, P1b, P2w, P2b, P3w, P3b):
    s = edge_index[0]
    d = edge_index[1]
    pad = EP - E
    sp = jnp.concatenate([s, jnp.zeros((pad,), i32)])
    dp = jnp.concatenate([d, jnp.full((pad,), N, i32)])
    zeros128 = jnp.zeros((NP, H), f32)
    zeros16 = jnp.zeros((NP, 16), f32)
    ones16 = jnp.ones((CH, 16), f32)

    degs = _sdeg(dp, ones16, zeros16)
    q1, dinv = _k2(x, W1, degs)
    acc1 = _sscat(q1, sp, dp, zeros128)
    zres = jnp.zeros((N, H), f32)
    h1, q2 = _ka(acc1, q1, zres, dinv, b1.reshape(1, H), W2)
    acc2 = _sscat(q2, sp, dp, zeros128)
    h2, q3 = _ka(acc2, q2, h1, dinv, b2.reshape(1, H), W3)
    acc3 = _sscat(q3, sp, dp, zeros128)
    hw, as16, ad16, m16 = _kb(acc3, q3, h2, dinv, b3.reshape(1, H), Wg,
                              att_src.reshape(1, H), att_dst.reshape(1, H))
    ee, dens = _satt(as16, ad16, m16, sp, dp, zeros16)
    gacc = _sgat(hw, ee, sp, dp, zeros128)
    out = _k7(gacc, hw, as16, ad16, m16, dens, bg.reshape(1, H),
              batch.reshape(N, 1), P1w,
              P1b.reshape(1, H // 2), P2w, P2b.reshape(1, H // 4), P3w,
              P3b.reshape(1, 1))
    return out.reshape(-1)


# speedup vs baseline: 1.2448x; 1.0032x over previous
import functools

import jax
import jax.numpy as jnp
from jax import lax
from jax.experimental import pallas as pl
from jax.experimental.pallas import tpu as pltpu
from jax.experimental.pallas import tpu_sc as plsc

N = 10000
E = 320000
H = 128
G = 64
HEADS = 4
DH = 32

NC = 2
NS = 16
NW = NC * NS
CH = 128
PTE = 10240
NCHUNK = PTE // CH
EP = NW * PTE
NP = 10112
STRIPE = NP // NS
KF_S, KS_S = 136, 24
KF_G, KS_G = 178, 142
NBLK = 10
BLK = N // NBLK

f32 = jnp.float32
i32 = jnp.int32

_mesh = plsc.VectorSubcoreMesh(core_axis_name="c", subcore_axis_name="s")
_untiled = pltpu.CompilerParams(use_tc_tiling_on_sc=False)


def _tile_ids():
    cid = lax.axis_index("c")
    sid = lax.axis_index("s")
    return cid, sid, cid * NS + sid


@functools.partial(
    pl.kernel,
    out_type=jax.ShapeDtypeStruct((NC, NP, 16), f32),
    mesh=_mesh,
    scratch_types=[
        pltpu.VMEM((CH,), i32),
        pltpu.VMEM((CH, 16), f32),
        pltpu.VMEM_SHARED((NP, 16), f32),
        pltpu.SemaphoreType.DMA,
    ],
)
def _sdeg(dp_hbm, ones_hbm, zeros_hbm, out_hbm, didx_v, ones_v, acc_sh, sem):
    cid, sid, wid = _tile_ids()
    pltpu.sync_copy(zeros_hbm.at[pl.ds(sid * STRIPE, STRIPE)],
                    acc_sh.at[pl.ds(sid * STRIPE, STRIPE)])
    pltpu.sync_copy(ones_hbm, ones_v)
    plsc.subcore_barrier()
    base = wid * PTE

    def step(j, c):
        pltpu.sync_copy(dp_hbm.at[pl.ds(base + j * CH, CH)], didx_v)
        pltpu.sync_copy(ones_v, acc_sh.at[didx_v], add=True)
        return c

    lax.fori_loop(0, NCHUNK, step, 0)
    plsc.subcore_barrier()
    pltpu.sync_copy(acc_sh.at[pl.ds(sid * STRIPE, STRIPE)],
                    out_hbm.at[cid, pl.ds(sid * STRIPE, STRIPE)])


@functools.partial(
    pl.kernel,
    out_type=jax.ShapeDtypeStruct((NC, NP, H), f32),
    mesh=_mesh,
    scratch_types=[
        pltpu.VMEM((CH,), i32),
        pltpu.VMEM((CH,), i32),
        pltpu.VMEM((CH,), i32),
        pltpu.VMEM((CH,), i32),
        pltpu.VMEM((CH, H), f32),
        pltpu.VMEM((CH, H), f32),
        pltpu.VMEM_SHARED((NP, H), f32),
        pltpu.SemaphoreType.DMA,
        pltpu.SemaphoreType.DMA,
    ],
)
def _sscat(q_hbm, sp_hbm, dp_hbm, zeros_hbm, out_hbm,
           sidx_a, sidx_b, didx_a, didx_b, rows_a, rows_b,
           acc_sh, sem_a, sem_b):
    cid, sid, wid = _tile_ids()
    pltpu.sync_copy(zeros_hbm.at[pl.ds(sid * STRIPE, STRIPE)],
                    acc_sh.at[pl.ds(sid * STRIPE, STRIPE)])
    plsc.subcore_barrier()
    base = jnp.where(cid == 0, sid * KF_S, NS * KF_S + sid * KS_S) * CH
    nch = jnp.where(cid == 0, KF_S, KS_S)

    @pl.when(nch > 0)
    def _():
        pltpu.sync_copy(sp_hbm.at[pl.ds(base, CH)], sidx_a)
        pltpu.sync_copy(dp_hbm.at[pl.ds(base, CH)], didx_a)
        pltpu.async_copy(q_hbm.at[sidx_a], rows_a, sem_a)

    def step(j2, c):
        o = base + 2 * j2 * CH
        pltpu.sync_copy(sp_hbm.at[pl.ds(o + CH, CH)], sidx_b)
        pltpu.sync_copy(dp_hbm.at[pl.ds(o + CH, CH)], didx_b)
        pltpu.make_async_copy(q_hbm.at[sidx_a], rows_a, sem_a).wait()
        pltpu.async_copy(q_hbm.at[sidx_b], rows_b, sem_b)
        pltpu.sync_copy(rows_a, acc_sh.at[didx_a], add=True)

        @pl.when(2 * j2 + 2 < nch)
        def _():
            pltpu.sync_copy(sp_hbm.at[pl.ds(o + 2 * CH, CH)], sidx_a)
            pltpu.sync_copy(dp_hbm.at[pl.ds(o + 2 * CH, CH)], didx_a)

        pltpu.make_async_copy(q_hbm.at[sidx_b], rows_b, sem_b).wait()

        @pl.when(2 * j2 + 2 < nch)
        def _():
            pltpu.async_copy(q_hbm.at[sidx_a], rows_a, sem_a)

        pltpu.sync_copy(rows_b, acc_sh.at[didx_b], add=True)
        return c

    lax.fori_loop(0, nch // 2, step, 0)
    plsc.subcore_barrier()
    pltpu.sync_copy(acc_sh.at[pl.ds(sid * STRIPE, STRIPE)],
                    out_hbm.at[cid, pl.ds(sid * STRIPE, STRIPE)])


@functools.partial(
    pl.kernel,
    out_type=(jax.ShapeDtypeStruct((EP, 16), f32),
              jax.ShapeDtypeStruct((NC, NP, 16), f32)),
    mesh=_mesh,
    compiler_params=_untiled,
    scratch_types=[
        pltpu.VMEM((CH,), i32),
        pltpu.VMEM((CH,), i32),
        pltpu.VMEM((CH,), i32),
        pltpu.VMEM((CH,), i32),
        pltpu.VMEM((CH, 16), f32),
        pltpu.VMEM((CH, 16), f32),
        pltpu.VMEM((CH, 16), f32),
        pltpu.VMEM((CH, 16), f32),
        pltpu.VMEM((CH, 16), f32),
        pltpu.VMEM((1, 16), f32),
        pltpu.VMEM_SHARED((NP, 16), f32),
        pltpu.SemaphoreType.DMA,
        pltpu.SemaphoreType.DMA,
    ],
)
def _satt(as_hbm, ad_hbm, m_hbm, sp_hbm, dp_hbm, zeros_hbm,
          ee_hbm, dens_hbm,
          sidx_a, sidx_b, didx_a, didx_b, as_a, as_b, ad_a, ad_b,
          ee_v, m_v, den_sh, sem_a, sem_b):
    cid, sid, wid = _tile_ids()
    pltpu.sync_copy(zeros_hbm.at[pl.ds(sid * STRIPE, STRIPE)],
                    den_sh.at[pl.ds(sid * STRIPE, STRIPE)])
    pltpu.sync_copy(m_hbm, m_v)
    plsc.subcore_barrier()
    base = wid * PTE
    pltpu.sync_copy(sp_hbm.at[pl.ds(base, CH)], sidx_a)
    pltpu.sync_copy(dp_hbm.at[pl.ds(base, CH)], didx_a)
    pltpu.async_copy(as_hbm.at[sidx_a], as_a, sem_a)
    pltpu.async_copy(ad_hbm.at[didx_a], ad_a, sem_a)

    def compute(as_v, ad_v):
        def inner(i, cc):
            m16 = m_v[0, :]
            a = as_v[i, :] + ad_v[i, :]
            ee_v[i, :] = jnp.exp(jnp.maximum(a, 0.2 * a) - m16)
            return cc

        lax.fori_loop(0, CH, inner, 0)

    def step(j2, c):
        off = base + 2 * j2 * CH
        pltpu.sync_copy(sp_hbm.at[pl.ds(off + CH, CH)], sidx_b)
        pltpu.sync_copy(dp_hbm.at[pl.ds(off + CH, CH)], didx_b)
        pltpu.make_async_copy(as_hbm.at[sidx_a], as_a, sem_a).wait()
        pltpu.make_async_copy(ad_hbm.at[didx_a], ad_a, sem_a).wait()
        pltpu.async_copy(as_hbm.at[sidx_b], as_b, sem_b)
        pltpu.async_copy(ad_hbm.at[didx_b], ad_b, sem_b)
        compute(as_a, ad_a)
        pltpu.sync_copy(ee_v, ee_hbm.at[pl.ds(off, CH)])
        pltpu.sync_copy(ee_v, den_sh.at[didx_a], add=True)

        @pl.when(2 * j2 + 2 < NCHUNK)
        def _():
            pltpu.sync_copy(sp_hbm.at[pl.ds(off + 2 * CH, CH)], sidx_a)
            pltpu.sync_copy(dp_hbm.at[pl.ds(off + 2 * CH, CH)], didx_a)

        pltpu.make_async_copy(as_hbm.at[sidx_b], as_b, sem_b).wait()
        pltpu.make_async_copy(ad_hbm.at[didx_b], ad_b, sem_b).wait()

        @pl.when(2 * j2 + 2 < NCHUNK)
        def _():
            pltpu.async_copy(as_hbm.at[sidx_a], as_a, sem_a)
            pltpu.async_copy(ad_hbm.at[didx_a], ad_a, sem_a)

        compute(as_b, ad_b)
        pltpu.sync_copy(ee_v, ee_hbm.at[pl.ds(off + CH, CH)])
        pltpu.sync_copy(ee_v, den_sh.at[didx_b], add=True)
        return c

    lax.fori_loop(0, NCHUNK // 2, step, 0)
    plsc.subcore_barrier()
    pltpu.sync_copy(den_sh.at[pl.ds(sid * STRIPE, STRIPE)],
                    dens_hbm.at[cid, pl.ds(sid * STRIPE, STRIPE)])


CHG = 64
NCHG = PTE // CHG


@functools.partial(
    pl.kernel,
    out_type=jax.ShapeDtypeStruct((NC, NP, H), f32),
    mesh=_mesh,
    scratch_types=[
        pltpu.VMEM((CHG,), i32),
        pltpu.VMEM((CHG,), i32),
        pltpu.VMEM((CHG,), i32),
        pltpu.VMEM((CHG,), i32),
        pltpu.VMEM((CHG, H), f32),
        pltpu.VMEM((CHG, H), f32),
        pltpu.VMEM((CHG, 16), f32),
        pltpu.VMEM((CHG, 16), f32),
        pltpu.VMEM_SHARED((NP, H), f32),
        pltpu.SemaphoreType.DMA,
        pltpu.SemaphoreType.DMA,
    ],
)
def _sgat(hw_hbm, ee_hbm, sp_hbm, dp_hbm, zeros_hbm, out_hbm,
          sidx_a, sidx_b, didx_a, didx_b, rows_a, rows_b, ee_a, ee_b,
          acc_sh, sem_a, sem_b):
    cid, sid, wid = _tile_ids()
    pltpu.sync_copy(zeros_hbm.at[pl.ds(sid * STRIPE, STRIPE)],
                    acc_sh.at[pl.ds(sid * STRIPE, STRIPE)])
    plsc.subcore_barrier()
    base = jnp.where(cid == 0, sid * KF_G, NS * KF_G + sid * KS_G) * CHG
    nch = jnp.where(cid == 0, KF_G, KS_G)
    pltpu.sync_copy(sp_hbm.at[pl.ds(base, CHG)], sidx_a)
    pltpu.sync_copy(dp_hbm.at[pl.ds(base, CHG)], didx_a)
    pltpu.async_copy(hw_hbm.at[sidx_a], rows_a, sem_a)
    pltpu.sync_copy(ee_hbm.at[pl.ds(base, CHG)], ee_a)

    def compute(rows_v, ee_v):
        def inner(i, cc):
            ee = ee_v[i, :]
            for h in range(HEADS):
                c_h = ee[h]
                rows_v[i, pl.ds(32 * h, 16)] = (
                    rows_v[i, pl.ds(32 * h, 16)] * c_h)
                rows_v[i, pl.ds(32 * h + 16, 16)] = (
                    rows_v[i, pl.ds(32 * h + 16, 16)] * c_h)
            return cc

        lax.fori_loop(0, CHG, inner, 0)

    def step(j2, c):
        off = base + 2 * j2 * CHG
        pltpu.sync_copy(sp_hbm.at[pl.ds(off + CHG, CHG)], sidx_b)
        pltpu.sync_copy(dp_hbm.at[pl.ds(off + CHG, CHG)], didx_b)
        pltpu.sync_copy(ee_hbm.at[pl.ds(off + CHG, CHG)], ee_b)
        pltpu.make_async_copy(hw_hbm.at[sidx_a], rows_a, sem_a).wait()
        pltpu.async_copy(hw_hbm.at[sidx_b], rows_b, sem_b)
        compute(rows_a, ee_a)
        pltpu.sync_copy(rows_a, acc_sh.at[didx_a], add=True)

        @pl.when(2 * j2 + 2 < nch)
        def _():
            pltpu.sync_copy(sp_hbm.at[pl.ds(off + 2 * CHG, CHG)], sidx_a)
            pltpu.sync_copy(dp_hbm.at[pl.ds(off + 2 * CHG, CHG)], didx_a)
            pltpu.sync_copy(ee_hbm.at[pl.ds(off + 2 * CHG, CHG)], ee_a)

        pltpu.make_async_copy(hw_hbm.at[sidx_b], rows_b, sem_b).wait()

        @pl.when(2 * j2 + 2 < nch)
        def _():
            pltpu.async_copy(hw_hbm.at[sidx_a], rows_a, sem_a)

        compute(rows_b, ee_b)
        pltpu.sync_copy(rows_b, acc_sh.at[didx_b], add=True)
        return c

    lax.fori_loop(0, nch // 2, step, 0)
    plsc.subcore_barrier()
    pltpu.sync_copy(acc_sh.at[pl.ds(sid * STRIPE, STRIPE)],
                    out_hbm.at[cid, pl.ds(sid * STRIPE, STRIPE)])


def _k2_body(x_ref, w_ref, degs_ref, q_ref, dinv_ref):
    degs = degs_ref[...]
    deg = degs[0, :, :1] + degs[1, :, :1] + 1.0
    dinv = lax.rsqrt(deg)
    dinv_ref[...] = dinv
    q_ref[...] = dinv * jnp.dot(x_ref[...], w_ref[...],
                                preferred_element_type=f32)


_k2 = pl.pallas_call(
    _k2_body,
    grid=(NBLK,),
    in_specs=[
        pl.BlockSpec((BLK, H), lambda i: (i, 0)),
        pl.BlockSpec((H, H), lambda i: (0, 0)),
        pl.BlockSpec((NC, BLK, 16), lambda i: (0, i, 0)),
    ],
    out_specs=[
        pl.BlockSpec((BLK, H), lambda i: (i, 0)),
        pl.BlockSpec((BLK, 1), lambda i: (i, 0)),
    ],
    out_shape=[
        jax.ShapeDtypeStruct((N, H), f32),
        jax.ShapeDtypeStruct((N, 1), f32),
    ],
)


def _ka_body(acc_ref, q_ref, res_ref, dinv_ref, b_ref, w_ref, h_ref, qn_ref):
    acc = acc_ref[...]
    dinv = dinv_ref[...]
    y = dinv * (acc[0] + acc[1] + q_ref[...]) + b_ref[...]
    h = res_ref[...] + jnp.maximum(y, 0.0)
    h_ref[...] = h
    qn_ref[...] = dinv * jnp.dot(h, w_ref[...], preferred_element_type=f32)


_ka = pl.pallas_call(
    _ka_body,
    grid=(NBLK,),
    in_specs=[
        pl.BlockSpec((NC, BLK, H), lambda i: (0, i, 0)),
        pl.BlockSpec((BLK, H), lambda i: (i, 0)),
        pl.BlockSpec((BLK, H), lambda i: (i, 0)),
        pl.BlockSpec((BLK, 1), lambda i: (i, 0)),
        pl.BlockSpec((1, H), lambda i: (0, 0)),
        pl.BlockSpec((H, H), lambda i: (0, 0)),
    ],
    out_specs=[
        pl.BlockSpec((BLK, H), lambda i: (i, 0)),
        pl.BlockSpec((BLK, H), lambda i: (i, 0)),
    ],
    out_shape=[
        jax.ShapeDtypeStruct((N, H), f32),
        jax.ShapeDtypeStruct((N, H), f32),
    ],
)


def _kb_body(acc_ref, q_ref, res_ref, dinv_ref, b_ref, wg_ref, atts_ref,
             attd_ref, hw_ref, as_ref, ad_ref, m_ref, ms_acc, md_acc):
    i = pl.program_id(0)
    acc = acc_ref[...]
    dinv = dinv_ref[...]
    y = dinv * (acc[0] + acc[1] + q_ref[...]) + b_ref[...]
    h = res_ref[...] + jnp.maximum(y, 0.0)
    hw = jnp.dot(h, wg_ref[...], preferred_element_type=f32)
    hw_ref[...] = hw
    lane = lax.broadcasted_iota(i32, (H, 16), 0)
    col = lax.broadcasted_iota(i32, (H, 16), 1)
    e16 = jnp.where((col < HEADS) & (lane // DH == col), 1.0, 0.0)
    as16 = jnp.dot(hw * atts_ref[...], e16, preferred_element_type=f32)
    ad16 = jnp.dot(hw * attd_ref[...], e16, preferred_element_type=f32)
    as_ref[...] = as16
    ad_ref[...] = ad16
    pms = jnp.max(as16, axis=0, keepdims=True)
    pmd = jnp.max(ad16, axis=0, keepdims=True)

    @pl.when(i == 0)
    def _():
        ms_acc[...] = pms
        md_acc[...] = pmd

    @pl.when(i > 0)
    def _():
        ms_acc[...] = jnp.maximum(ms_acc[...], pms)
        md_acc[...] = jnp.maximum(md_acc[...], pmd)

    @pl.when(i == NBLK - 1)
    def _():
        m_ref[...] = ms_acc[...] + md_acc[...]


_kb = pl.pallas_call(
    _kb_body,
    grid=(NBLK,),
    in_specs=[
        pl.BlockSpec((NC, BLK, H), lambda i: (0, i, 0)),
        pl.BlockSpec((BLK, H), lambda i: (i, 0)),
        pl.BlockSpec((BLK, H), lambda i: (i, 0)),
        pl.BlockSpec((BLK, 1), lambda i: (i, 0)),
        pl.BlockSpec((1, H), lambda i: (0, 0)),
        pl.BlockSpec((H, H), lambda i: (0, 0)),
        pl.BlockSpec((1, H), lambda i: (0, 0)),
        pl.BlockSpec((1, H), lambda i: (0, 0)),
    ],
    out_specs=[
        pl.BlockSpec((BLK, H), lambda i: (i, 0)),
        pl.BlockSpec((BLK, 16), lambda i: (i, 0)),
        pl.BlockSpec((BLK, 16), lambda i: (i, 0)),
        pl.BlockSpec((1, 16), lambda i: (0, 0)),
    ],
    out_shape=[
        jax.ShapeDtypeStruct((N, H), f32),
        jax.ShapeDtypeStruct((N, 16), f32),
        jax.ShapeDtypeStruct((N, 16), f32),
        jax.ShapeDtypeStruct((1, 16), f32),
    ],
    scratch_shapes=[
        pltpu.VMEM((1, 16), f32),
        pltpu.VMEM((1, 16), f32),
    ],
)


def _k7_body(gacc_ref, hw_ref, as_ref, ad_ref, m_ref, dens_ref, bg_ref,
             bcol_ref, p1w_ref, p1b_ref, p2w_ref, p2b_ref,
             p3w_ref, p3b_ref, out_ref, gm_acc, gx_acc, cnt_acc):
    i = pl.program_id(0)

    @pl.when(i == 0)
    def _():
        gm_acc[...] = jnp.zeros((G, H), f32)
        gx_acc[...] = jnp.full((G, H), -jnp.inf, f32)
        cnt_acc[...] = jnp.zeros((G, 1), f32)

    a = as_ref[...] + ad_ref[...]
    ee_self = jnp.exp(jnp.maximum(a, 0.2 * a) - m_ref[...])
    dens = dens_ref[...]
    den16 = dens[0] + dens[1] + ee_self
    colc = lax.broadcasted_iota(i32, (16, H), 0)
    lanec = lax.broadcasted_iota(i32, (16, H), 1)
    t16 = jnp.where((colc < HEADS) & (lanec // DH == colc), 1.0, 0.0)
    den128 = jnp.dot(den16, t16, preferred_element_type=f32)
    ee128 = jnp.dot(ee_self, t16, preferred_element_type=f32)
    gacc = gacc_ref[...]
    g = (gacc[0] + gacc[1] + ee128 * hw_ref[...]) / den128 + bg_ref[...]

    bcol = bcol_ref[...]
    grow = lax.broadcasted_iota(i32, (1, G), 1)
    onehot = jnp.where(bcol == grow, 1.0, 0.0)
    dn = (((0,), (0,)), ((), ()))
    gm_acc[...] = gm_acc[...] + lax.dot_general(
        onehot, g, dn, preferred_element_type=f32)
    cnt_acc[...] = cnt_acc[...] + lax.dot_general(
        onehot, jnp.ones((BLK, 1), f32), dn, preferred_element_type=f32)

    g0 = bcol_ref[0, 0]
    g1 = bcol_ref[BLK - 1, 0]

    def gbody(gi, c):
        vals = jnp.where(bcol == gi, g, -jnp.inf)
        m = jnp.max(vals, axis=0, keepdims=True)
        cur = gx_acc[pl.ds(gi, 1), :]
        gx_acc[pl.ds(gi, 1), :] = jnp.maximum(cur, m)
        return c

    lax.fori_loop(g0, g1 + 1, gbody, 0)

    @pl.when(i == NBLK - 1)
    def _():
        cnt = jnp.maximum(cnt_acc[...], 1.0)
        z = jnp.concatenate([gm_acc[...] / cnt, gx_acc[...]], axis=1)
        z = jnp.maximum(jnp.dot(z, p1w_ref[...], preferred_element_type=f32)
                        + p1b_ref[...], 0.0)
        z = jnp.maximum(jnp.dot(z, p2w_ref[...], preferred_element_type=f32)
                        + p2b_ref[...], 0.0)
        out_ref[...] = (jnp.dot(z, p3w_ref[...], preferred_element_type=f32)
                        + p3b_ref[...])


_k7 = pl.pallas_call(
    _k7_body,
    grid=(NBLK,),
    in_specs=[
        pl.BlockSpec((NC, BLK, H), lambda i: (0, i, 0)),
        pl.BlockSpec((BLK, H), lambda i: (i, 0)),
        pl.BlockSpec((BLK, 16), lambda i: (i, 0)),
        pl.BlockSpec((BLK, 16), lambda i: (i, 0)),
        pl.BlockSpec((1, 16), lambda i: (0, 0)),
        pl.BlockSpec((NC, BLK, 16), lambda i: (0, i, 0)),
        pl.BlockSpec((1, H), lambda i: (0, 0)),
        pl.BlockSpec((BLK, 1), lambda i: (i, 0)),
        pl.BlockSpec((2 * H, H // 2), lambda i: (0, 0)),
        pl.BlockSpec((1, H // 2), lambda i: (0, 0)),
        pl.BlockSpec((H // 2, H // 4), lambda i: (0, 0)),
        pl.BlockSpec((1, H // 4), lambda i: (0, 0)),
        pl.BlockSpec((H // 4, 1), lambda i: (0, 0)),
        pl.BlockSpec((1, 1), lambda i: (0, 0)),
    ],
    out_specs=pl.BlockSpec((G, 1), lambda i: (0, 0)),
    out_shape=jax.ShapeDtypeStruct((G, 1), f32),
    scratch_shapes=[
        pltpu.VMEM((G, H), f32),
        pltpu.VMEM((G, H), f32),
        pltpu.VMEM((G, 1), f32),
    ],
)


def kernel(x, edge_index, batch, W1, b1, W2, b2, W3, b3, Wg, att_src,
           att_dst, bg, P1w, P1b, P2w, P2b, P3w, P3b):
    s = edge_index[0]
    d = edge_index[1]
    pad = EP - E
    sp = jnp.concatenate([s, jnp.zeros((pad,), i32)])
    dp = jnp.concatenate([d, jnp.full((pad,), N, i32)])
    zeros128 = jnp.zeros((NP, H), f32)
    zeros16 = jnp.zeros((NP, 16), f32)
    ones16 = jnp.ones((CH, 16), f32)

    degs = _sdeg(dp, ones16, zeros16)
    q1, dinv = _k2(x, W1, degs)
    acc1 = _sscat(q1, sp, dp, zeros128)
    zres = jnp.zeros((N, H), f32)
    h1, q2 = _ka(acc1, q1, zres, dinv, b1.reshape(1, H), W2)
    acc2 = _sscat(q2, sp, dp, zeros128)
    h2, q3 = _ka(acc2, q2, h1, dinv, b2.reshape(1, H), W3)
    acc3 = _sscat(q3, sp, dp, zeros128)
    hw, as16, ad16, m16 = _kb(acc3, q3, h2, dinv, b3.reshape(1, H), Wg,
                              att_src.reshape(1, H), att_dst.reshape(1, H))
    ee, dens = _satt(as16, ad16, m16, sp, dp, zeros16)
    gacc = _sgat(hw, ee, sp, dp, zeros128)
    out = _k7(gacc, hw, as16, ad16, m16, dens, bg.reshape(1, H),
              batch.reshape(N, 1), P1w,
              P1b.reshape(1, H // 2), P2w, P2b.reshape(1, H // 4), P3w,
              P3b.reshape(1, 1))
    return out.reshape(-1)


# speedup vs baseline: 1.3388x; 1.0755x over previous
import functools

import jax
import jax.numpy as jnp
from jax import lax
from jax.experimental import pallas as pl
from jax.experimental.pallas import tpu as pltpu
from jax.experimental.pallas import tpu_sc as plsc

N = 10000
E = 320000
H = 128
G = 64
HEADS = 4
DH = 32

NC = 2
NS = 16
NW = NC * NS
CH = 128
PTE = 10240
NCHUNK = PTE // CH
EP = NW * PTE
NP = 10112
STRIPE = NP // NS
KF_S, KS_S = 136, 24
KF_G, KS_G = 178, 142
NBLK = 10
BLK = N // NBLK

f32 = jnp.float32
i32 = jnp.int32

_mesh = plsc.VectorSubcoreMesh(core_axis_name="c", subcore_axis_name="s")
_untiled = pltpu.CompilerParams(use_tc_tiling_on_sc=False)


def _tile_ids():
    cid = lax.axis_index("c")
    sid = lax.axis_index("s")
    return cid, sid, cid * NS + sid


@functools.partial(
    pl.kernel,
    out_type=jax.ShapeDtypeStruct((NC, NP, 16), f32),
    mesh=_mesh,
    scratch_types=[
        pltpu.VMEM((CH,), i32),
        pltpu.VMEM((CH, 16), f32),
        pltpu.VMEM_SHARED((NP, 16), f32),
        pltpu.SemaphoreType.DMA,
    ],
)
def _sdeg(dp_hbm, ones_hbm, zeros_hbm, out_hbm, didx_v, ones_v, acc_sh, sem):
    cid, sid, wid = _tile_ids()
    pltpu.sync_copy(zeros_hbm.at[pl.ds(sid * STRIPE, STRIPE)],
                    acc_sh.at[pl.ds(sid * STRIPE, STRIPE)])
    pltpu.sync_copy(ones_hbm, ones_v)
    plsc.subcore_barrier()
    base = wid * PTE

    def step(j, c):
        pltpu.sync_copy(dp_hbm.at[pl.ds(base + j * CH, CH)], didx_v)
        pltpu.sync_copy(ones_v, acc_sh.at[didx_v], add=True)
        return c

    lax.fori_loop(0, NCHUNK, step, 0)
    plsc.subcore_barrier()
    pltpu.sync_copy(acc_sh.at[pl.ds(sid * STRIPE, STRIPE)],
                    out_hbm.at[cid, pl.ds(sid * STRIPE, STRIPE)])


@functools.partial(
    pl.kernel,
    out_type=jax.ShapeDtypeStruct((NC, NP, H), f32),
    mesh=_mesh,
    scratch_types=[
        pltpu.VMEM((CH,), i32),
        pltpu.VMEM((CH,), i32),
        pltpu.VMEM((CH,), i32),
        pltpu.VMEM((CH,), i32),
        pltpu.VMEM((CH, H), f32),
        pltpu.VMEM((CH, H), f32),
        pltpu.VMEM_SHARED((NP, H), f32),
        pltpu.SemaphoreType.DMA,
        pltpu.SemaphoreType.DMA,
    ],
)
def _sscat(q_hbm, sp_hbm, dp_hbm, zeros_hbm, out_hbm,
           sidx_a, sidx_b, didx_a, didx_b, rows_a, rows_b,
           acc_sh, sem_a, sem_b):
    cid, sid, wid = _tile_ids()
    pltpu.sync_copy(zeros_hbm.at[pl.ds(sid * STRIPE, STRIPE)],
                    acc_sh.at[pl.ds(sid * STRIPE, STRIPE)])
    plsc.subcore_barrier()
    base = jnp.where(cid == 0, sid * KF_S, NS * KF_S + sid * KS_S) * CH
    nch = jnp.where(cid == 0, KF_S, KS_S)

    @pl.when(nch > 0)
    def _():
        pltpu.sync_copy(sp_hbm.at[pl.ds(base, CH)], sidx_a)
        pltpu.sync_copy(dp_hbm.at[pl.ds(base, CH)], didx_a)
        pltpu.async_copy(q_hbm.at[sidx_a], rows_a, sem_a)

    def step(j2, c):
        o = base + 2 * j2 * CH
        pltpu.sync_copy(sp_hbm.at[pl.ds(o + CH, CH)], sidx_b)
        pltpu.sync_copy(dp_hbm.at[pl.ds(o + CH, CH)], didx_b)
        pltpu.make_async_copy(q_hbm.at[sidx_a], rows_a, sem_a).wait()
        pltpu.async_copy(q_hbm.at[sidx_b], rows_b, sem_b)
        pltpu.sync_copy(rows_a, acc_sh.at[didx_a], add=True)

        @pl.when(2 * j2 + 2 < nch)
        def _():
            pltpu.sync_copy(sp_hbm.at[pl.ds(o + 2 * CH, CH)], sidx_a)
            pltpu.sync_copy(dp_hbm.at[pl.ds(o + 2 * CH, CH)], didx_a)

        pltpu.make_async_copy(q_hbm.at[sidx_b], rows_b, sem_b).wait()

        @pl.when(2 * j2 + 2 < nch)
        def _():
            pltpu.async_copy(q_hbm.at[sidx_a], rows_a, sem_a)

        pltpu.sync_copy(rows_b, acc_sh.at[didx_b], add=True)
        return c

    lax.fori_loop(0, nch // 2, step, 0)
    plsc.subcore_barrier()
    pltpu.sync_copy(acc_sh.at[pl.ds(sid * STRIPE, STRIPE)],
                    out_hbm.at[cid, pl.ds(sid * STRIPE, STRIPE)])


@functools.partial(
    pl.kernel,
    out_type=(jax.ShapeDtypeStruct((EP, 16), f32),
              jax.ShapeDtypeStruct((NC, NP, 16), f32)),
    mesh=_mesh,
    compiler_params=_untiled,
    scratch_types=[
        pltpu.VMEM((CH,), i32),
        pltpu.VMEM((CH,), i32),
        pltpu.VMEM((CH,), i32),
        pltpu.VMEM((CH,), i32),
        pltpu.VMEM((CH, 16), f32),
        pltpu.VMEM((CH, 16), f32),
        pltpu.VMEM((CH, 16), f32),
        pltpu.VMEM((CH, 16), f32),
        pltpu.VMEM((CH, 16), f32),
        pltpu.VMEM((1, 16), f32),
        pltpu.VMEM_SHARED((NP, 16), f32),
        pltpu.SemaphoreType.DMA,
        pltpu.SemaphoreType.DMA,
    ],
)
def _satt(as_hbm, ad_hbm, m_hbm, sp_hbm, dp_hbm, zeros_hbm,
          ee_hbm, dens_hbm,
          sidx_a, sidx_b, didx_a, didx_b, as_a, as_b, ad_a, ad_b,
          ee_v, m_v, den_sh, sem_a, sem_b):
    cid, sid, wid = _tile_ids()
    pltpu.sync_copy(zeros_hbm.at[pl.ds(sid * STRIPE, STRIPE)],
                    den_sh.at[pl.ds(sid * STRIPE, STRIPE)])
    pltpu.sync_copy(m_hbm, m_v)
    plsc.subcore_barrier()
    base = wid * PTE
    pltpu.sync_copy(sp_hbm.at[pl.ds(base, CH)], sidx_a)
    pltpu.sync_copy(dp_hbm.at[pl.ds(base, CH)], didx_a)
    pltpu.async_copy(as_hbm.at[sidx_a], as_a, sem_a)
    pltpu.async_copy(ad_hbm.at[didx_a], ad_a, sem_a)

    def compute(as_v, ad_v):
        def inner(i, cc):
            m16 = m_v[0, :]
            a = as_v[i, :] + ad_v[i, :]
            ee_v[i, :] = jnp.exp(jnp.maximum(a, 0.2 * a) - m16)
            return cc

        lax.fori_loop(0, CH, inner, 0)

    def step(j2, c):
        off = base + 2 * j2 * CH
        pltpu.sync_copy(sp_hbm.at[pl.ds(off + CH, CH)], sidx_b)
        pltpu.sync_copy(dp_hbm.at[pl.ds(off + CH, CH)], didx_b)
        pltpu.make_async_copy(as_hbm.at[sidx_a], as_a, sem_a).wait()
        pltpu.make_async_copy(ad_hbm.at[didx_a], ad_a, sem_a).wait()
        pltpu.async_copy(as_hbm.at[sidx_b], as_b, sem_b)
        pltpu.async_copy(ad_hbm.at[didx_b], ad_b, sem_b)
        compute(as_a, ad_a)
        pltpu.sync_copy(ee_v, ee_hbm.at[pl.ds(off, CH)])
        pltpu.sync_copy(ee_v, den_sh.at[didx_a], add=True)

        @pl.when(2 * j2 + 2 < NCHUNK)
        def _():
            pltpu.sync_copy(sp_hbm.at[pl.ds(off + 2 * CH, CH)], sidx_a)
            pltpu.sync_copy(dp_hbm.at[pl.ds(off + 2 * CH, CH)], didx_a)

        pltpu.make_async_copy(as_hbm.at[sidx_b], as_b, sem_b).wait()
        pltpu.make_async_copy(ad_hbm.at[didx_b], ad_b, sem_b).wait()

        @pl.when(2 * j2 + 2 < NCHUNK)
        def _():
            pltpu.async_copy(as_hbm.at[sidx_a], as_a, sem_a)
            pltpu.async_copy(ad_hbm.at[didx_a], ad_a, sem_a)

        compute(as_b, ad_b)
        pltpu.sync_copy(ee_v, ee_hbm.at[pl.ds(off + CH, CH)])
        pltpu.sync_copy(ee_v, den_sh.at[didx_b], add=True)
        return c

    lax.fori_loop(0, NCHUNK // 2, step, 0)
    plsc.subcore_barrier()
    pltpu.sync_copy(den_sh.at[pl.ds(sid * STRIPE, STRIPE)],
                    dens_hbm.at[cid, pl.ds(sid * STRIPE, STRIPE)])


CHG = 64
NCHG = PTE // CHG


@functools.partial(
    pl.kernel,
    out_type=jax.ShapeDtypeStruct((NC, NP, H), f32),
    mesh=_mesh,
    compiler_params=_untiled,
    scratch_types=[
        pltpu.VMEM((CHG,), i32),
        pltpu.VMEM((CHG,), i32),
        pltpu.VMEM((CHG,), i32),
        pltpu.VMEM((CHG,), i32),
        pltpu.VMEM((CHG, H), f32),
        pltpu.VMEM((CHG, H), f32),
        pltpu.VMEM((CHG, 16), f32),
        pltpu.VMEM((CHG, 16), f32),
        pltpu.VMEM_SHARED((NP, H), f32),
        pltpu.SemaphoreType.DMA,
        pltpu.SemaphoreType.DMA,
    ],
)
def _sgat(hw_hbm, ee_hbm, sp_hbm, dp_hbm, zeros_hbm, out_hbm,
          sidx_a, sidx_b, didx_a, didx_b, rows_a, rows_b, ee_a, ee_b,
          acc_sh, sem_a, sem_b):
    cid, sid, wid = _tile_ids()
    pltpu.sync_copy(zeros_hbm.at[pl.ds(sid * STRIPE, STRIPE)],
                    acc_sh.at[pl.ds(sid * STRIPE, STRIPE)])
    plsc.subcore_barrier()
    base = jnp.where(cid == 0, sid * KF_G, NS * KF_G + sid * KS_G) * CHG
    nch = jnp.where(cid == 0, KF_G, KS_G)
    pltpu.sync_copy(sp_hbm.at[pl.ds(base, CHG)], sidx_a)
    pltpu.sync_copy(dp_hbm.at[pl.ds(base, CHG)], didx_a)
    pltpu.async_copy(hw_hbm.at[sidx_a], rows_a, sem_a)
    pltpu.sync_copy(ee_hbm.at[pl.ds(base, CHG)], ee_a)

    def compute(rows_v, ee_v):
        def inner(i, cc):
            ee = ee_v[i, :]
            for h in range(HEADS):
                c_h = ee[h]
                rows_v[i, pl.ds(32 * h, 16)] = (
                    rows_v[i, pl.ds(32 * h, 16)] * c_h)
                rows_v[i, pl.ds(32 * h + 16, 16)] = (
                    rows_v[i, pl.ds(32 * h + 16, 16)] * c_h)
            return cc

        lax.fori_loop(0, CHG, inner, 0)

    def step(j2, c):
        off = base + 2 * j2 * CHG
        pltpu.sync_copy(sp_hbm.at[pl.ds(off + CHG, CHG)], sidx_b)
        pltpu.sync_copy(dp_hbm.at[pl.ds(off + CHG, CHG)], didx_b)
        pltpu.sync_copy(ee_hbm.at[pl.ds(off + CHG, CHG)], ee_b)
        pltpu.make_async_copy(hw_hbm.at[sidx_a], rows_a, sem_a).wait()
        pltpu.async_copy(hw_hbm.at[sidx_b], rows_b, sem_b)
        compute(rows_a, ee_a)
        pltpu.sync_copy(rows_a, acc_sh.at[didx_a], add=True)

        @pl.when(2 * j2 + 2 < nch)
        def _():
            pltpu.sync_copy(sp_hbm.at[pl.ds(off + 2 * CHG, CHG)], sidx_a)
            pltpu.sync_copy(dp_hbm.at[pl.ds(off + 2 * CHG, CHG)], didx_a)
            pltpu.sync_copy(ee_hbm.at[pl.ds(off + 2 * CHG, CHG)], ee_a)

        pltpu.make_async_copy(hw_hbm.at[sidx_b], rows_b, sem_b).wait()

        @pl.when(2 * j2 + 2 < nch)
        def _():
            pltpu.async_copy(hw_hbm.at[sidx_a], rows_a, sem_a)

        compute(rows_b, ee_b)
        pltpu.sync_copy(rows_b, acc_sh.at[didx_b], add=True)
        return c

    lax.fori_loop(0, nch // 2, step, 0)
    plsc.subcore_barrier()
    pltpu.sync_copy(acc_sh.at[pl.ds(sid * STRIPE, STRIPE)],
                    out_hbm.at[cid, pl.ds(sid * STRIPE, STRIPE)])


def _k2_body(x_ref, w_ref, degs_ref, q_ref, dinv_ref):
    degs = degs_ref[...]
    deg = degs[0, :, :1] + degs[1, :, :1] + 1.0
    dinv = lax.rsqrt(deg)
    dinv_ref[...] = dinv
    q_ref[...] = dinv * jnp.dot(x_ref[...], w_ref[...],
                                preferred_element_type=f32)


_k2 = pl.pallas_call(
    _k2_body,
    grid=(NBLK,),
    in_specs=[
        pl.BlockSpec((BLK, H), lambda i: (i, 0)),
        pl.BlockSpec((H, H), lambda i: (0, 0)),
        pl.BlockSpec((NC, BLK, 16), lambda i: (0, i, 0)),
    ],
    out_specs=[
        pl.BlockSpec((BLK, H), lambda i: (i, 0)),
        pl.BlockSpec((BLK, 1), lambda i: (i, 0)),
    ],
    out_shape=[
        jax.ShapeDtypeStruct((N, H), f32),
        jax.ShapeDtypeStruct((N, 1), f32),
    ],
)


def _ka_body(acc_ref, q_ref, res_ref, dinv_ref, b_ref, w_ref, h_ref, qn_ref):
    acc = acc_ref[...]
    dinv = dinv_ref[...]
    y = dinv * (acc[0] + acc[1] + q_ref[...]) + b_ref[...]
    h = res_ref[...] + jnp.maximum(y, 0.0)
    h_ref[...] = h
    qn_ref[...] = dinv * jnp.dot(h, w_ref[...], preferred_element_type=f32)


_ka = pl.pallas_call(
    _ka_body,
    grid=(NBLK,),
    in_specs=[
        pl.BlockSpec((NC, BLK, H), lambda i: (0, i, 0)),
        pl.BlockSpec((BLK, H), lambda i: (i, 0)),
        pl.BlockSpec((BLK, H), lambda i: (i, 0)),
        pl.BlockSpec((BLK, 1), lambda i: (i, 0)),
        pl.BlockSpec((1, H), lambda i: (0, 0)),
        pl.BlockSpec((H, H), lambda i: (0, 0)),
    ],
    out_specs=[
        pl.BlockSpec((BLK, H), lambda i: (i, 0)),
        pl.BlockSpec((BLK, H), lambda i: (i, 0)),
    ],
    out_shape=[
        jax.ShapeDtypeStruct((N, H), f32),
        jax.ShapeDtypeStruct((N, H), f32),
    ],
)


def _kb_body(acc_ref, q_ref, res_ref, dinv_ref, b_ref, wg_ref, atts_ref,
             attd_ref, hw_ref, as_ref, ad_ref, m_ref, ms_acc, md_acc):
    i = pl.program_id(0)
    acc = acc_ref[...]
    dinv = dinv_ref[...]
    y = dinv * (acc[0] + acc[1] + q_ref[...]) + b_ref[...]
    h = res_ref[...] + jnp.maximum(y, 0.0)
    hw = jnp.dot(h, wg_ref[...], preferred_element_type=f32)
    hw_ref[...] = hw
    lane = lax.broadcasted_iota(i32, (H, 16), 0)
    col = lax.broadcasted_iota(i32, (H, 16), 1)
    e16 = jnp.where((col < HEADS) & (lane // DH == col), 1.0, 0.0)
    as16 = jnp.dot(hw * atts_ref[...], e16, preferred_element_type=f32)
    ad16 = jnp.dot(hw * attd_ref[...], e16, preferred_element_type=f32)
    as_ref[...] = as16
    ad_ref[...] = ad16
    pms = jnp.max(as16, axis=0, keepdims=True)
    pmd = jnp.max(ad16, axis=0, keepdims=True)

    @pl.when(i == 0)
    def _():
        ms_acc[...] = pms
        md_acc[...] = pmd

    @pl.when(i > 0)
    def _():
        ms_acc[...] = jnp.maximum(ms_acc[...], pms)
        md_acc[...] = jnp.maximum(md_acc[...], pmd)

    @pl.when(i == NBLK - 1)
    def _():
        m_ref[...] = ms_acc[...] + md_acc[...]


_kb = pl.pallas_call(
    _kb_body,
    grid=(NBLK,),
    in_specs=[
        pl.BlockSpec((NC, BLK, H), lambda i: (0, i, 0)),
        pl.BlockSpec((BLK, H), lambda i: (i, 0)),
        pl.BlockSpec((BLK, H), lambda i: (i, 0)),
        pl.BlockSpec((BLK, 1), lambda i: (i, 0)),
        pl.BlockSpec((1, H), lambda i: (0, 0)),
        pl.BlockSpec((H, H), lambda i: (0, 0)),
        pl.BlockSpec((1, H), lambda i: (0, 0)),
        pl.BlockSpec((1, H), lambda i: (0, 0)),
    ],
    out_specs=[
        pl.BlockSpec((BLK, H), lambda i: (i, 0)),
        pl.BlockSpec((BLK, 16), lambda i: (i, 0)),
        pl.BlockSpec((BLK, 16), lambda i: (i, 0)),
        pl.BlockSpec((1, 16), lambda i: (0, 0)),
    ],
    out_shape=[
        jax.ShapeDtypeStruct((N, H), f32),
        jax.ShapeDtypeStruct((N, 16), f32),
        jax.ShapeDtypeStruct((N, 16), f32),
        jax.ShapeDtypeStruct((1, 16), f32),
    ],
    scratch_shapes=[
        pltpu.VMEM((1, 16), f32),
        pltpu.VMEM((1, 16), f32),
    ],
)


def _k7_body(gacc_ref, hw_ref, as_ref, ad_ref, m_ref, dens_ref, bg_ref,
             bcol_ref, p1w_ref, p1b_ref, p2w_ref, p2b_ref,
             p3w_ref, p3b_ref, out_ref, gm_acc, gx_acc, cnt_acc):
    i = pl.program_id(0)

    @pl.when(i == 0)
    def _():
        gm_acc[...] = jnp.zeros((G, H), f32)
        gx_acc[...] = jnp.full((G, H), -jnp.inf, f32)
        cnt_acc[...] = jnp.zeros((G, 1), f32)

    a = as_ref[...] + ad_ref[...]
    ee_self = jnp.exp(jnp.maximum(a, 0.2 * a) - m_ref[...])
    dens = dens_ref[...]
    den16 = dens[0] + dens[1] + ee_self
    colc = lax.broadcasted_iota(i32, (16, H), 0)
    lanec = lax.broadcasted_iota(i32, (16, H), 1)
    t16 = jnp.where((colc < HEADS) & (lanec // DH == colc), 1.0, 0.0)
    den128 = jnp.dot(den16, t16, preferred_element_type=f32)
    ee128 = jnp.dot(ee_self, t16, preferred_element_type=f32)
    gacc = gacc_ref[...]
    g = (gacc[0] + gacc[1] + ee128 * hw_ref[...]) / den128 + bg_ref[...]

    bcol = bcol_ref[...]
    grow = lax.broadcasted_iota(i32, (1, G), 1)
    onehot = jnp.where(bcol == grow, 1.0, 0.0)
    dn = (((0,), (0,)), ((), ()))
    gm_acc[...] = gm_acc[...] + lax.dot_general(
        onehot, g, dn, preferred_element_type=f32)
    cnt_acc[...] = cnt_acc[...] + lax.dot_general(
        onehot, jnp.ones((BLK, 1), f32), dn, preferred_element_type=f32)

    g0 = bcol_ref[0, 0]
    g1 = bcol_ref[BLK - 1, 0]

    def gbody(gi, c):
        vals = jnp.where(bcol == gi, g, -jnp.inf)
        m = jnp.max(vals, axis=0, keepdims=True)
        cur = gx_acc[pl.ds(gi, 1), :]
        gx_acc[pl.ds(gi, 1), :] = jnp.maximum(cur, m)
        return c

    lax.fori_loop(g0, g1 + 1, gbody, 0)

    @pl.when(i == NBLK - 1)
    def _():
        cnt = jnp.maximum(cnt_acc[...], 1.0)
        z = jnp.concatenate([gm_acc[...] / cnt, gx_acc[...]], axis=1)
        z = jnp.maximum(jnp.dot(z, p1w_ref[...], preferred_element_type=f32)
                        + p1b_ref[...], 0.0)
        z = jnp.maximum(jnp.dot(z, p2w_ref[...], preferred_element_type=f32)
                        + p2b_ref[...], 0.0)
        out_ref[...] = (jnp.dot(z, p3w_ref[...], preferred_element_type=f32)
                        + p3b_ref[...])


_k7 = pl.pallas_call(
    _k7_body,
    grid=(NBLK,),
    in_specs=[
        pl.BlockSpec((NC, BLK, H), lambda i: (0, i, 0)),
        pl.BlockSpec((BLK, H), lambda i: (i, 0)),
        pl.BlockSpec((BLK, 16), lambda i: (i, 0)),
        pl.BlockSpec((BLK, 16), lambda i: (i, 0)),
        pl.BlockSpec((1, 16), lambda i: (0, 0)),
        pl.BlockSpec((NC, BLK, 16), lambda i: (0, i, 0)),
        pl.BlockSpec((1, H), lambda i: (0, 0)),
        pl.BlockSpec((BLK, 1), lambda i: (i, 0)),
        pl.BlockSpec((2 * H, H // 2), lambda i: (0, 0)),
        pl.BlockSpec((1, H // 2), lambda i: (0, 0)),
        pl.BlockSpec((H // 2, H // 4), lambda i: (0, 0)),
        pl.BlockSpec((1, H // 4), lambda i: (0, 0)),
        pl.BlockSpec((H // 4, 1), lambda i: (0, 0)),
        pl.BlockSpec((1, 1), lambda i: (0, 0)),
    ],
    out_specs=pl.BlockSpec((G, 1), lambda i: (0, 0)),
    out_shape=jax.ShapeDtypeStruct((G, 1), f32),
    scratch_shapes=[
        pltpu.VMEM((G, H), f32),
        pltpu.VMEM((G, H), f32),
        pltpu.VMEM((G, 1), f32),
    ],
)


def kernel(x, edge_index, batch, W1, b1, W2, b2, W3, b3, Wg, att_src,
           att_dst, bg, P1w, P1b, P2w, P2b, P3w, P3b):
    s = edge_index[0]
    d = edge_index[1]
    pad = EP - E
    sp = jnp.concatenate([s, jnp.zeros((pad,), i32)])
    dp = jnp.concatenate([d, jnp.full((pad,), N, i32)])
    zeros128 = jnp.zeros((NP, H), f32)
    zeros16 = jnp.zeros((NP, 16), f32)
    ones16 = jnp.ones((CH, 16), f32)

    degs = _sdeg(dp, ones16, zeros16)
    q1, dinv = _k2(x, W1, degs)
    acc1 = _sscat(q1, sp, dp, zeros128)
    zres = jnp.zeros((N, H), f32)
    h1, q2 = _ka(acc1, q1, zres, dinv, b1.reshape(1, H), W2)
    acc2 = _sscat(q2, sp, dp, zeros128)
    h2, q3 = _ka(acc2, q2, h1, dinv, b2.reshape(1, H), W3)
    acc3 = _sscat(q3, sp, dp, zeros128)
    hw, as16, ad16, m16 = _kb(acc3, q3, h2, dinv, b3.reshape(1, H), Wg,
                              att_src.reshape(1, H), att_dst.reshape(1, H))
    ee, dens = _satt(as16, ad16, m16, sp, dp, zeros16)
    zeros128u = jnp.zeros((NP, H), f32) + 0.0
    gacc = _sgat(hw, ee, sp, dp, zeros128u)
    out = _k7(gacc, hw, as16, ad16, m16, dens, bg.reshape(1, H),
              batch.reshape(N, 1), P1w,
              P1b.reshape(1, H // 2), P2w, P2b.reshape(1, H // 4), P3w,
              P3b.reshape(1, 1))
    return out.reshape(-1)
